# Initial kernel scaffold; baseline (speedup 1.0000x reference)
import jax
import jax.numpy as jnp
from jax.experimental import pallas as pl


def kernel(x, edge_index, edge_attr, xe1, xe2, ee1, ee2, W1, b1, W2, b2, gamma, beta):
    raise NotImplementedError("write your pallas kernel here")


# speedup vs baseline: 3.0462x; 3.0462x over previous
import functools

import jax
import jax.numpy as jnp
from jax import lax
from jax.experimental import pallas as pl
from jax.experimental.pallas import tpu as pltpu
from jax.experimental.pallas import tpu_sc as plsc

N = 10000
E = 320000
D = 128
L = 5

NC = 2
NS = 16
NW = NC * NS
Np = 10240
Ep = NW * Np
EPT = Ep // NW
ECH = 128
NCHUNK = EPT // ECH
NPT = Np // NW
NNCH = 64
NNCHUNK = NPT // NNCH
RPT = Np // NS

_mesh = plsc.VectorSubcoreMesh(core_axis_name="c", subcore_axis_name="s")


def _precompute_body(comb_hbm, oh128_hbm, x01_hbm, c3_hbm, dst3_hbm, z128_hbm,
                     h0_out, cnt_out,
                     nidx_v, nbuf_v, cidx_v, didx_v, ohbuf_v, cnt_sh, sem):
    c = lax.axis_index("c")
    s = lax.axis_index("s")
    w = c * NS + s
    pltpu.sync_copy(x01_hbm.at[w], nidx_v)
    for j in range(NNCHUNK):
        pltpu.async_copy(comb_hbm.at[nidx_v.at[j]], nbuf_v, sem).wait()
        pltpu.sync_copy(nbuf_v, h0_out.at[pl.ds(w * NPT + j * NNCH, NNCH)])
    pltpu.sync_copy(c3_hbm.at[w], cidx_v)
    pltpu.sync_copy(dst3_hbm.at[w], didx_v)
    rows = pl.ds(s * RPT, RPT)
    pltpu.sync_copy(z128_hbm.at[rows], cnt_sh.at[rows])
    plsc.subcore_barrier()

    def body(j, carry):
        pltpu.async_copy(oh128_hbm.at[cidx_v.at[j]], ohbuf_v, sem).wait()
        pltpu.sync_copy(ohbuf_v, cnt_sh.at[didx_v.at[j]], add=True)
        return carry

    lax.fori_loop(0, NCHUNK, body, 0)
    plsc.subcore_barrier()
    pltpu.sync_copy(cnt_sh.at[rows], cnt_out.at[c, rows])


_precompute = functools.partial(
    pl.kernel,
    _precompute_body,
    out_type=(
        jax.ShapeDtypeStruct((Np, D), jnp.float32),
        jax.ShapeDtypeStruct((NC, Np, D), jnp.float32),
    ),
    mesh=_mesh,
    scratch_types=[
        pltpu.VMEM((NNCHUNK, NNCH), jnp.int32),
        pltpu.VMEM((NNCH, D), jnp.float32),
        pltpu.VMEM((NCHUNK, ECH), jnp.int32),
        pltpu.VMEM((NCHUNK, ECH), jnp.int32),
        pltpu.VMEM((ECH, D), jnp.float32),
        pltpu.VMEM_SHARED((Np, D), jnp.float32),
        pltpu.SemaphoreType.DMA,
    ],
)()


def _agg_body(h_hbm, src3_hbm, dst3_hbm, z128_hbm,
              agg_out,
              sidx_v, didx_v, rows_v, agg_sh, sem):
    c = lax.axis_index("c")
    s = lax.axis_index("s")
    w = c * NS + s
    pltpu.sync_copy(src3_hbm.at[w], sidx_v)
    pltpu.sync_copy(dst3_hbm.at[w], didx_v)
    rows = pl.ds(s * RPT, RPT)

    @pl.when(c == 0)
    def _():
        pltpu.sync_copy(h_hbm.at[rows], agg_sh.at[rows])

    @pl.when(c == 1)
    def _():
        pltpu.sync_copy(z128_hbm.at[rows], agg_sh.at[rows])

    plsc.subcore_barrier()

    def body(j, carry):
        pltpu.async_copy(h_hbm.at[sidx_v.at[j]], rows_v, sem).wait()
        pltpu.sync_copy(rows_v, agg_sh.at[didx_v.at[j]], add=True)
        return carry

    lax.fori_loop(0, NCHUNK, body, 0)
    plsc.subcore_barrier()
    pltpu.sync_copy(agg_sh.at[rows], agg_out.at[c, rows])


_agg = functools.partial(
    pl.kernel,
    _agg_body,
    out_type=jax.ShapeDtypeStruct((NC, Np, D), jnp.float32),
    mesh=_mesh,
    scratch_types=[
        pltpu.VMEM((NCHUNK, ECH), jnp.int32),
        pltpu.VMEM((NCHUNK, ECH), jnp.int32),
        pltpu.VMEM((ECH, D), jnp.float32),
        pltpu.VMEM_SHARED((Np, D), jnp.float32),
        pltpu.SemaphoreType.DMA,
    ],
)()


RBLK = 512


def _mlp_body(relu, agg_ref, cnt_ref, cls_ref, w1_ref, b1_ref, w2_ref, b2_ref,
              g_ref, be_ref, sr_ref, out_ref):
    z = (agg_ref[0] + agg_ref[1]
         + jnp.dot(cnt_ref[0] + cnt_ref[1], cls_ref[...],
                   preferred_element_type=jnp.float32)
         + sr_ref[...])
    m = jnp.maximum(jnp.dot(z, w1_ref[...],
                            preferred_element_type=jnp.float32) + b1_ref[...],
                    0.0)
    o = jnp.dot(m, w2_ref[...], preferred_element_type=jnp.float32) + b2_ref[...]
    o = o * g_ref[...] + be_ref[...]
    out_ref[...] = jnp.maximum(o, 0.0) if relu else o


def _mlp(relu, agg2, cnt2, cls16, w1t, b1r, w2t, b2r, gr, ber, srr):
    grid = (Np // RBLK,)
    full = lambda shape: pl.BlockSpec(shape, lambda i: (0,) * len(shape))
    return pl.pallas_call(
        functools.partial(_mlp_body, relu),
        grid=grid,
        in_specs=[
            pl.BlockSpec((NC, RBLK, D), lambda i: (0, i, 0)),
            pl.BlockSpec((NC, RBLK, 16), lambda i: (0, i, 0)),
            full((16, D)),
            full((D, 2 * D)),
            full((1, 2 * D)),
            full((2 * D, D)),
            full((1, D)),
            full((1, D)),
            full((1, D)),
            full((1, D)),
        ],
        out_specs=pl.BlockSpec((RBLK, D), lambda i: (i, 0)),
        out_shape=jax.ShapeDtypeStruct((Np, D), jnp.float32),
    )(agg2, cnt2, cls16, w1t, b1r, w2t, b2r, gr, ber, srr)


def kernel(x, edge_index, edge_attr, xe1, xe2, ee1, ee2, W1, b1, W2, b2,
           gamma, beta):
    eps = 1e-5
    f32 = jnp.float32
    src = edge_index[0].astype(jnp.int32)
    dst = edge_index[1].astype(jnp.int32)
    pad_e = jnp.full((Ep - E,), Np - 1, jnp.int32)
    src3 = jnp.concatenate([src, pad_e]).reshape(NW, NCHUNK, ECH)
    dst_p = jnp.concatenate([dst, pad_e])
    dst3 = dst_p.reshape(NW, NCHUNK, ECH)
    c_e = (edge_attr[:, 0] * 3 + edge_attr[:, 1]).astype(jnp.int32)
    c3 = jnp.concatenate([c_e, jnp.zeros((Ep - E,), jnp.int32)]
                         ).reshape(NW, NCHUNK, ECH)
    x01 = (x[:, 0] * 3 + x[:, 1]).astype(jnp.int32)
    x01_3 = jnp.concatenate([x01, jnp.zeros((Np - N,), jnp.int32)]
                            ).reshape(NW, NNCHUNK, NNCH)
    comb = (xe1[:, None, :] + xe2[None, :3, :]).reshape(-1, D)
    oh128 = jnp.eye(16, D, dtype=f32)
    z128 = jnp.zeros((Np, D), f32)
    gsc = (gamma / jnp.sqrt(1.0 + eps)).astype(f32)

    h, cnt128 = _precompute(comb, oh128, x01_3, c3, dst3, z128)
    cnt2 = cnt128[:, :, :16]
    for l in range(L):
        cls9 = (ee1[l, :3, None, :] + ee2[l, None, :3, :]).reshape(9, D)
        cls16 = jnp.concatenate([cls9, jnp.zeros((7, D), f32)], 0)
        srr = (ee1[l, 4] + ee2[l, 0]).reshape(1, D)
        agg2 = _agg(h, src3, dst3, z128)
        h = _mlp(l < L - 1, agg2, cnt2, cls16,
                 W1[l].T, b1[l].reshape(1, 2 * D),
                 W2[l].T, b2[l].reshape(1, D),
                 gsc[l].reshape(1, D), beta[l].reshape(1, D), srr)
    return h[:N]


# speedup vs baseline: 5.4473x; 1.7882x over previous
import functools

import jax
import jax.numpy as jnp
from jax import lax
from jax.experimental import pallas as pl
from jax.experimental.pallas import tpu as pltpu
from jax.experimental.pallas import tpu_sc as plsc

N = 10000
E = 320000
D = 128
L = 5

NC = 2
NS = 16
NW = NC * NS
Np = 10240
Ep = NW * Np
EPT = Ep // NW
ECH = 128
NCHUNK = EPT // ECH
NPT = Np // NW
NNCH = 64
NNCHUNK = NPT // NNCH
RPT = Np // NS
HC = NCHUNK // 2

_mesh = plsc.VectorSubcoreMesh(core_axis_name="c", subcore_axis_name="s")


def _gs_pipeline(src_tab, idx3_hbm, didx3_hbm, w, sidx_v, didx_v,
                 buf0_v, buf1_v, acc_sh, semA, semB):
    for p in range(2):
        pltpu.sync_copy(idx3_hbm.at[w, pl.ds(p * HC, HC)], sidx_v)
        pltpu.sync_copy(didx3_hbm.at[w, pl.ds(p * HC, HC)], didx_v)
        pltpu.async_copy(src_tab.at[sidx_v.at[0]], buf0_v, semA)

        def body(jj, carry):
            j0 = 2 * jj
            d1 = pltpu.async_copy(src_tab.at[sidx_v.at[j0 + 1]], buf1_v, semB)
            pltpu.make_async_copy(src_tab.at[sidx_v.at[j0]], buf0_v,
                                  semA).wait()
            pltpu.sync_copy(buf0_v, acc_sh.at[didx_v.at[j0]], add=True)

            @pl.when(jj < HC // 2 - 1)
            def _():
                pltpu.async_copy(src_tab.at[sidx_v.at[j0 + 2]], buf0_v, semA)

            d1.wait()
            pltpu.sync_copy(buf1_v, acc_sh.at[didx_v.at[j0 + 1]], add=True)
            return carry

        lax.fori_loop(0, HC // 2, body, 0)


def _precompute_body(comb_hbm, oh128_hbm, x01_hbm, c3_hbm, dst3_hbm, z128_hbm,
                     h0_out, cnt_out,
                     nidx_v, cidx_v, didx_v, oh0_v, oh1_v, cnt_sh,
                     semA, semB):
    c = lax.axis_index("c")
    s = lax.axis_index("s")
    w = c * NS + s
    pltpu.sync_copy(x01_hbm.at[w], nidx_v)
    nbuf_v = oh0_v.at[pl.ds(0, NNCH)]
    for j in range(NNCHUNK):
        pltpu.async_copy(comb_hbm.at[nidx_v.at[j]], nbuf_v, semA).wait()
        pltpu.sync_copy(nbuf_v, h0_out.at[pl.ds(w * NPT + j * NNCH, NNCH)])
    rows = pl.ds(s * RPT, RPT)
    pltpu.sync_copy(z128_hbm.at[rows], cnt_sh.at[rows])
    plsc.subcore_barrier()
    _gs_pipeline(oh128_hbm, c3_hbm, dst3_hbm, w, cidx_v, didx_v,
                 oh0_v, oh1_v, cnt_sh, semA, semB)
    plsc.subcore_barrier()
    pltpu.sync_copy(cnt_sh.at[rows], cnt_out.at[c, rows])


_precompute = functools.partial(
    pl.kernel,
    _precompute_body,
    out_type=(
        jax.ShapeDtypeStruct((Np, D), jnp.float32),
        jax.ShapeDtypeStruct((NC, Np, D), jnp.float32),
    ),
    mesh=_mesh,
    scratch_types=[
        pltpu.VMEM((NNCHUNK, NNCH), jnp.int32),
        pltpu.VMEM((HC, ECH), jnp.int32),
        pltpu.VMEM((HC, ECH), jnp.int32),
        pltpu.VMEM((ECH, D), jnp.float32),
        pltpu.VMEM((ECH, D), jnp.float32),
        pltpu.VMEM_SHARED((Np, D), jnp.float32),
        pltpu.SemaphoreType.DMA,
        pltpu.SemaphoreType.DMA,
    ],
)()


def _agg_body(h_hbm, src3_hbm, dst3_hbm, z128_hbm,
              agg_out,
              sidx_v, didx_v, rows0_v, rows1_v, agg_sh, semA, semB):
    c = lax.axis_index("c")
    s = lax.axis_index("s")
    w = c * NS + s
    rows = pl.ds(s * RPT, RPT)

    @pl.when(c == 0)
    def _():
        pltpu.sync_copy(h_hbm.at[rows], agg_sh.at[rows])

    @pl.when(c == 1)
    def _():
        pltpu.sync_copy(z128_hbm.at[rows], agg_sh.at[rows])

    plsc.subcore_barrier()
    _gs_pipeline(h_hbm, src3_hbm, dst3_hbm, w, sidx_v, didx_v,
                 rows0_v, rows1_v, agg_sh, semA, semB)
    plsc.subcore_barrier()
    pltpu.sync_copy(agg_sh.at[rows], agg_out.at[c, rows])


_agg = functools.partial(
    pl.kernel,
    _agg_body,
    out_type=jax.ShapeDtypeStruct((NC, Np, D), jnp.float32),
    mesh=_mesh,
    scratch_types=[
        pltpu.VMEM((HC, ECH), jnp.int32),
        pltpu.VMEM((HC, ECH), jnp.int32),
        pltpu.VMEM((ECH, D), jnp.float32),
        pltpu.VMEM((ECH, D), jnp.float32),
        pltpu.VMEM_SHARED((Np, D), jnp.float32),
        pltpu.SemaphoreType.DMA,
        pltpu.SemaphoreType.DMA,
    ],
)()


RBLK = 512


def _mlp_body(relu, agg_ref, cnt_ref, cls_ref, w1_ref, b1_ref, w2_ref, b2_ref,
              g_ref, be_ref, sr_ref, out_ref):
    z = (agg_ref[0] + agg_ref[1]
         + jnp.dot(cnt_ref[0] + cnt_ref[1], cls_ref[...],
                   preferred_element_type=jnp.float32)
         + sr_ref[...])
    m = jnp.maximum(jnp.dot(z, w1_ref[...],
                            preferred_element_type=jnp.float32) + b1_ref[...],
                    0.0)
    o = jnp.dot(m, w2_ref[...], preferred_element_type=jnp.float32) + b2_ref[...]
    o = o * g_ref[...] + be_ref[...]
    out_ref[...] = jnp.maximum(o, 0.0) if relu else o


def _mlp(relu, agg2, cnt2, cls16, w1t, b1r, w2t, b2r, gr, ber, srr):
    grid = (Np // RBLK,)
    full = lambda shape: pl.BlockSpec(shape, lambda i: (0,) * len(shape))
    return pl.pallas_call(
        functools.partial(_mlp_body, relu),
        grid=grid,
        in_specs=[
            pl.BlockSpec((NC, RBLK, D), lambda i: (0, i, 0)),
            pl.BlockSpec((NC, RBLK, 16), lambda i: (0, i, 0)),
            full((16, D)),
            full((D, 2 * D)),
            full((1, 2 * D)),
            full((2 * D, D)),
            full((1, D)),
            full((1, D)),
            full((1, D)),
            full((1, D)),
        ],
        out_specs=pl.BlockSpec((RBLK, D), lambda i: (i, 0)),
        out_shape=jax.ShapeDtypeStruct((Np, D), jnp.float32),
    )(agg2, cnt2, cls16, w1t, b1r, w2t, b2r, gr, ber, srr)


def kernel(x, edge_index, edge_attr, xe1, xe2, ee1, ee2, W1, b1, W2, b2,
           gamma, beta):
    eps = 1e-5
    f32 = jnp.float32
    src = edge_index[0].astype(jnp.int32)
    dst = edge_index[1].astype(jnp.int32)
    pad_e = jnp.full((Ep - E,), Np - 1, jnp.int32)
    src3 = jnp.concatenate([src, pad_e]).reshape(NW, NCHUNK, ECH)
    dst_p = jnp.concatenate([dst, pad_e])
    dst3 = dst_p.reshape(NW, NCHUNK, ECH)
    c_e = (edge_attr[:, 0] * 3 + edge_attr[:, 1]).astype(jnp.int32)
    c3 = jnp.concatenate([c_e, jnp.zeros((Ep - E,), jnp.int32)]
                         ).reshape(NW, NCHUNK, ECH)
    c3 = c3 + (jnp.arange(NW, dtype=jnp.int32) * 16)[:, None, None]
    x01 = (x[:, 0] * 3 + x[:, 1]).astype(jnp.int32)
    x01_3 = jnp.concatenate([x01, jnp.zeros((Np - N,), jnp.int32)]
                            ).reshape(NW, NNCHUNK, NNCH)
    comb = (xe1[:, None, :] + xe2[None, :3, :]).reshape(-1, D)
    oh128 = jnp.tile(jnp.eye(16, D, dtype=f32), (NW, 1))
    z128 = jnp.zeros((Np, D), f32)
    gsc = (gamma / jnp.sqrt(1.0 + eps)).astype(f32)

    h, cnt128 = _precompute(comb, oh128, x01_3, c3, dst3, z128)
    cnt2 = cnt128[:, :, :16]
    for l in range(L):
        cls9 = (ee1[l, :3, None, :] + ee2[l, None, :3, :]).reshape(9, D)
        cls16 = jnp.concatenate([cls9, jnp.zeros((7, D), f32)], 0)
        srr = (ee1[l, 4] + ee2[l, 0]).reshape(1, D)
        agg2 = _agg(h, src3, dst3, z128)
        h = _mlp(l < L - 1, agg2, cnt2, cls16,
                 W1[l].T, b1[l].reshape(1, 2 * D),
                 W2[l].T, b2[l].reshape(1, D),
                 gsc[l].reshape(1, D), beta[l].reshape(1, D), srr)
    return h[:N]


# speedup vs baseline: 10.9799x; 2.0157x over previous
import functools

import jax
import jax.numpy as jnp
from jax import lax
from jax.experimental import pallas as pl
from jax.experimental.pallas import tpu as pltpu
from jax.experimental.pallas import tpu_sc as plsc

N = 10000
E = 320000
D = 128
L = 5

NC = 2
NS = 16
NW = NC * NS
Np = 10240
Ep = NW * Np
EPT = Ep // NW
ECH = 128
NCHUNK = EPT // ECH
NPT = Np // NW
NNCH = 64
NNCHUNK = NPT // NNCH
RPT = Np // NS
HC = NCHUNK // 2

_mesh = plsc.VectorSubcoreMesh(core_axis_name="c", subcore_axis_name="s")


def _gs_pipeline(src_tab, idx3_hbm, didx3_hbm, w, sidx_v, didx_v,
                 buf0_v, buf1_v, acc_sh, semA, semB):
    for p in range(2):
        pltpu.sync_copy(idx3_hbm.at[w, pl.ds(p * HC, HC)], sidx_v)
        pltpu.sync_copy(didx3_hbm.at[w, pl.ds(p * HC, HC)], didx_v)
        pltpu.async_copy(src_tab.at[sidx_v.at[0]], buf0_v, semA)

        def body(jj, carry):
            j0 = 2 * jj
            d1 = pltpu.async_copy(src_tab.at[sidx_v.at[j0 + 1]], buf1_v, semB)
            pltpu.make_async_copy(src_tab.at[sidx_v.at[j0]], buf0_v,
                                  semA).wait()
            pltpu.sync_copy(buf0_v, acc_sh.at[didx_v.at[j0]], add=True)

            @pl.when(jj < HC // 2 - 1)
            def _():
                pltpu.async_copy(src_tab.at[sidx_v.at[j0 + 2]], buf0_v, semA)

            d1.wait()
            pltpu.sync_copy(buf1_v, acc_sh.at[didx_v.at[j0 + 1]], add=True)
            return carry

        lax.fori_loop(0, HC // 2, body, 0)


def _precompute_body(comb_hbm, oh128_hbm, x01_hbm, c3_hbm, dst3_hbm, z128_hbm,
                     h0_out, cnt_out,
                     nidx_v, cidx_v, didx_v, oh0_v, oh1_v, cnt_sh,
                     semA, semB):
    c = lax.axis_index("c")
    s = lax.axis_index("s")
    w = c * NS + s
    pltpu.sync_copy(x01_hbm.at[w], nidx_v)
    nbuf_v = oh0_v.at[pl.ds(0, NNCH)]
    for j in range(NNCHUNK):
        pltpu.async_copy(comb_hbm.at[nidx_v.at[j]], nbuf_v, semA).wait()
        pltpu.sync_copy(nbuf_v, h0_out.at[pl.ds(w * NPT + j * NNCH, NNCH)])
    rows = pl.ds(s * RPT, RPT)
    pltpu.sync_copy(z128_hbm.at[rows], cnt_sh.at[rows])
    plsc.subcore_barrier()
    _gs_pipeline(oh128_hbm, c3_hbm, dst3_hbm, w, cidx_v, didx_v,
                 oh0_v, oh1_v, cnt_sh, semA, semB)
    plsc.subcore_barrier()
    pltpu.sync_copy(cnt_sh.at[rows], cnt_out.at[c, rows])


_precompute = functools.partial(
    pl.kernel,
    _precompute_body,
    out_type=(
        jax.ShapeDtypeStruct((Np, D), jnp.float32),
        jax.ShapeDtypeStruct((NC, Np, D), jnp.float32),
    ),
    mesh=_mesh,
    scratch_types=[
        pltpu.VMEM((NNCHUNK, NNCH), jnp.int32),
        pltpu.VMEM((HC, ECH), jnp.int32),
        pltpu.VMEM((HC, ECH), jnp.int32),
        pltpu.VMEM((ECH, D), jnp.float32),
        pltpu.VMEM((ECH, D), jnp.float32),
        pltpu.VMEM_SHARED((Np, D), jnp.float32),
        pltpu.SemaphoreType.DMA,
        pltpu.SemaphoreType.DMA,
    ],
)()


def _agg_body(h_hbm, src3_hbm, dst3_hbm, z128_hbm,
              agg_out,
              sidx_v, didx_v, rows0_v, rows1_v, agg_sh, semA, semB):
    c = lax.axis_index("c")
    s = lax.axis_index("s")
    w = c * NS + s
    rows = pl.ds(s * RPT, RPT)

    @pl.when(c == 0)
    def _():
        pltpu.sync_copy(h_hbm.at[rows], agg_sh.at[rows])

    @pl.when(c == 1)
    def _():
        pltpu.sync_copy(z128_hbm.at[rows], agg_sh.at[rows])

    plsc.subcore_barrier()
    _gs_pipeline(h_hbm, src3_hbm, dst3_hbm, w, sidx_v, didx_v,
                 rows0_v, rows1_v, agg_sh, semA, semB)
    plsc.subcore_barrier()
    pltpu.sync_copy(agg_sh.at[rows], agg_out.at[c, rows])


_agg = functools.partial(
    pl.kernel,
    _agg_body,
    out_type=jax.ShapeDtypeStruct((NC, Np, D), jnp.float32),
    mesh=_mesh,
    scratch_types=[
        pltpu.VMEM((HC, ECH), jnp.int32),
        pltpu.VMEM((HC, ECH), jnp.int32),
        pltpu.VMEM((ECH, D), jnp.float32),
        pltpu.VMEM((ECH, D), jnp.float32),
        pltpu.VMEM_SHARED((Np, D), jnp.float32),
        pltpu.SemaphoreType.DMA,
        pltpu.SemaphoreType.DMA,
    ],
)()


RBLK = 512


def _mlp_body(relu, agg_ref, cnt_ref, cls_ref, w1_ref, b1_ref, w2_ref, b2_ref,
              g_ref, be_ref, sr_ref, out_ref):
    z = (agg_ref[0] + agg_ref[1]
         + jnp.dot(cnt_ref[0] + cnt_ref[1], cls_ref[...],
                   preferred_element_type=jnp.float32)
         + sr_ref[...])
    m = jnp.maximum(jnp.dot(z, w1_ref[...],
                            preferred_element_type=jnp.float32) + b1_ref[...],
                    0.0)
    o = jnp.dot(m, w2_ref[...], preferred_element_type=jnp.float32) + b2_ref[...]
    o = o * g_ref[...] + be_ref[...]
    out_ref[...] = jnp.maximum(o, 0.0) if relu else o


def _mlp(relu, agg2, cnt2, cls16, w1t, b1r, w2t, b2r, gr, ber, srr):
    grid = (Np // RBLK,)
    full = lambda shape: pl.BlockSpec(shape, lambda i: (0,) * len(shape))
    return pl.pallas_call(
        functools.partial(_mlp_body, relu),
        grid=grid,
        in_specs=[
            pl.BlockSpec((NC, RBLK, D), lambda i: (0, i, 0)),
            pl.BlockSpec((NC, RBLK, 16), lambda i: (0, i, 0)),
            full((16, D)),
            full((D, 2 * D)),
            full((1, 2 * D)),
            full((2 * D, D)),
            full((1, D)),
            full((1, D)),
            full((1, D)),
            full((1, D)),
        ],
        out_specs=pl.BlockSpec((RBLK, D), lambda i: (i, 0)),
        out_shape=jax.ShapeDtypeStruct((Np, D), jnp.float32),
    )(agg2, cnt2, cls16, w1t, b1r, w2t, b2r, gr, ber, srr)


def kernel(x, edge_index, edge_attr, xe1, xe2, ee1, ee2, W1, b1, W2, b2,
           gamma, beta):
    eps = 1e-5
    f32 = jnp.float32
    src = edge_index[0].astype(jnp.int32)
    dst = edge_index[1].astype(jnp.int32)
    pad_e = N + jnp.arange(Ep - E, dtype=jnp.int32) % (Np - N)
    src3 = jnp.concatenate([src, pad_e]).reshape(NW, NCHUNK, ECH)
    dst_p = jnp.concatenate([dst, pad_e])
    dst3 = dst_p.reshape(NW, NCHUNK, ECH)
    c_e = (edge_attr[:, 0] * 3 + edge_attr[:, 1]).astype(jnp.int32)
    c3 = jnp.concatenate([c_e, jnp.zeros((Ep - E,), jnp.int32)]
                         ).reshape(NW, NCHUNK, ECH)
    c3 = c3 + (jnp.arange(NW, dtype=jnp.int32) * 16)[:, None, None]
    x01 = (x[:, 0] * 3 + x[:, 1]).astype(jnp.int32)
    x01_3 = jnp.concatenate([x01, jnp.zeros((Np - N,), jnp.int32)]
                            ).reshape(NW, NNCHUNK, NNCH)
    comb = (xe1[:, None, :] + xe2[None, :3, :]).reshape(-1, D)
    oh128 = jnp.tile(jnp.eye(16, D, dtype=f32), (NW, 1))
    z128 = jnp.zeros((Np, D), f32)
    gsc = (gamma / jnp.sqrt(1.0 + eps)).astype(f32)

    h, cnt128 = _precompute(comb, oh128, x01_3, c3, dst3, z128)
    cnt2 = cnt128[:, :, :16]
    for l in range(L):
        cls9 = (ee1[l, :3, None, :] + ee2[l, None, :3, :]).reshape(9, D)
        cls16 = jnp.concatenate([cls9, jnp.zeros((7, D), f32)], 0)
        srr = (ee1[l, 4] + ee2[l, 0]).reshape(1, D)
        agg2 = _agg(h, src3, dst3, z128)
        h = _mlp(l < L - 1, agg2, cnt2, cls16,
                 W1[l].T, b1[l].reshape(1, 2 * D),
                 W2[l].T, b2[l].reshape(1, D),
                 gsc[l].reshape(1, D), beta[l].reshape(1, D), srr)
    return h[:N]


# speedup vs baseline: 14.0094x; 1.2759x over previous
import functools

import jax
import jax.numpy as jnp
from jax import lax
from jax.experimental import pallas as pl
from jax.experimental.pallas import tpu as pltpu
from jax.experimental.pallas import tpu_sc as plsc

N = 10000
E = 320000
D = 128
L = 5

NC = 2
NS = 16
NW = NC * NS
Np = 10240
Ep = NW * Np
EPT = Ep // NW
ECH = 128
NCHUNK = EPT // ECH
NPT = Np // NW
NNCH = 64
NNCHUNK = NPT // NNCH
RPT = Np // NS
HC = NCHUNK // 2

_mesh = plsc.VectorSubcoreMesh(core_axis_name="c", subcore_axis_name="s")


def _gs_pipeline(src_tab, idx3_hbm, didx3_hbm, w, sidx_v, didx_v,
                 buf0_v, buf1_v, acc_sh, semA, semB):
    for p in range(2):
        pltpu.sync_copy(idx3_hbm.at[w, pl.ds(p * HC, HC)], sidx_v)
        pltpu.sync_copy(didx3_hbm.at[w, pl.ds(p * HC, HC)], didx_v)
        pltpu.async_copy(src_tab.at[sidx_v.at[0]], buf0_v, semA)

        def body(jj, carry):
            j0 = 2 * jj
            d1 = pltpu.async_copy(src_tab.at[sidx_v.at[j0 + 1]], buf1_v, semB)
            pltpu.make_async_copy(src_tab.at[sidx_v.at[j0]], buf0_v,
                                  semA).wait()
            pltpu.sync_copy(buf0_v, acc_sh.at[didx_v.at[j0]], add=True)

            @pl.when(jj < HC // 2 - 1)
            def _():
                pltpu.async_copy(src_tab.at[sidx_v.at[j0 + 2]], buf0_v, semA)

            d1.wait()
            pltpu.sync_copy(buf1_v, acc_sh.at[didx_v.at[j0 + 1]], add=True)
            return carry

        lax.fori_loop(0, HC // 2, body, 0)


def _precompute_body(comb_hbm, oh128_hbm, x01_hbm, c3_hbm, dst3_hbm, z128_hbm,
                     h0_out, cnt_out,
                     nidx_v, cidx_v, didx_v, oh0_v, oh1_v, cnt_sh,
                     semA, semB):
    c = lax.axis_index("c")
    s = lax.axis_index("s")
    w = c * NS + s
    pltpu.sync_copy(x01_hbm.at[w], nidx_v)
    nbuf_v = oh0_v.at[pl.ds(0, NNCH)]
    for j in range(NNCHUNK):
        pltpu.async_copy(comb_hbm.at[nidx_v.at[j]], nbuf_v, semA).wait()
        pltpu.sync_copy(nbuf_v, h0_out.at[pl.ds(w * NPT + j * NNCH, NNCH)])
    rows = pl.ds(s * RPT, RPT)
    pltpu.sync_copy(z128_hbm.at[rows], cnt_sh.at[rows])
    plsc.subcore_barrier()
    _gs_pipeline(oh128_hbm, c3_hbm, dst3_hbm, w, cidx_v, didx_v,
                 oh0_v, oh1_v, cnt_sh, semA, semB)
    plsc.subcore_barrier()
    pltpu.sync_copy(cnt_sh.at[rows], cnt_out.at[c, rows])


_precompute = functools.partial(
    pl.kernel,
    _precompute_body,
    out_type=(
        jax.ShapeDtypeStruct((Np, D), jnp.float32),
        jax.ShapeDtypeStruct((NC, Np, D), jnp.float32),
    ),
    mesh=_mesh,
    scratch_types=[
        pltpu.VMEM((NNCHUNK, NNCH), jnp.int32),
        pltpu.VMEM((HC, ECH), jnp.int32),
        pltpu.VMEM((HC, ECH), jnp.int32),
        pltpu.VMEM((ECH, D), jnp.float32),
        pltpu.VMEM((ECH, D), jnp.float32),
        pltpu.VMEM_SHARED((Np, D), jnp.float32),
        pltpu.SemaphoreType.DMA,
        pltpu.SemaphoreType.DMA,
    ],
)()


def _agg_body(h_hbm, src3_hbm, dst3_hbm, z128_hbm,
              agg_out,
              sidx_v, didx_v, rows0_v, rows1_v, agg_sh, semA, semB):
    c = lax.axis_index("c")
    s = lax.axis_index("s")
    w = c * NS + s
    rows = pl.ds(s * RPT, RPT)

    @pl.when(c == 0)
    def _():
        pltpu.sync_copy(h_hbm.at[rows], agg_sh.at[rows])

    @pl.when(c == 1)
    def _():
        pltpu.sync_copy(z128_hbm.at[rows], agg_sh.at[rows])

    plsc.subcore_barrier()
    _gs_pipeline(h_hbm, src3_hbm, dst3_hbm, w, sidx_v, didx_v,
                 rows0_v, rows1_v, agg_sh, semA, semB)
    plsc.subcore_barrier()
    pltpu.sync_copy(agg_sh.at[rows], agg_out.at[c, rows])


_agg = functools.partial(
    pl.kernel,
    _agg_body,
    out_type=jax.ShapeDtypeStruct((NC, Np, D), jnp.float32),
    mesh=_mesh,
    scratch_types=[
        pltpu.VMEM((HC, ECH), jnp.int32),
        pltpu.VMEM((HC, ECH), jnp.int32),
        pltpu.VMEM((ECH, D), jnp.float32),
        pltpu.VMEM((ECH, D), jnp.float32),
        pltpu.VMEM_SHARED((Np, D), jnp.float32),
        pltpu.SemaphoreType.DMA,
        pltpu.SemaphoreType.DMA,
    ],
)()


RBLK = 512


def _mlp_body(relu, agg_ref, cnt_ref, cls_ref, w1_ref, b1_ref, w2_ref, b2_ref,
              g_ref, be_ref, sr_ref, out_ref):
    z = (agg_ref[0] + agg_ref[1]
         + jnp.dot(cnt_ref[0] + cnt_ref[1], cls_ref[...],
                   preferred_element_type=jnp.float32)
         + sr_ref[...])
    m = jnp.maximum(jnp.dot(z, w1_ref[...],
                            preferred_element_type=jnp.float32) + b1_ref[...],
                    0.0)
    o = jnp.dot(m, w2_ref[...], preferred_element_type=jnp.float32) + b2_ref[...]
    o = o * g_ref[...] + be_ref[...]
    out_ref[...] = jnp.maximum(o, 0.0) if relu else o


def _mlp(relu, agg2, cnt2, cls16, w1t, b1r, w2t, b2r, gr, ber, srr):
    grid = (Np // RBLK,)
    full = lambda shape: pl.BlockSpec(shape, lambda i: (0,) * len(shape))
    return pl.pallas_call(
        functools.partial(_mlp_body, relu),
        grid=grid,
        in_specs=[
            pl.BlockSpec((NC, RBLK, D), lambda i: (0, i, 0)),
            pl.BlockSpec((NC, RBLK, 16), lambda i: (0, i, 0)),
            full((16, D)),
            full((D, 2 * D)),
            full((1, 2 * D)),
            full((2 * D, D)),
            full((1, D)),
            full((1, D)),
            full((1, D)),
            full((1, D)),
        ],
        out_specs=pl.BlockSpec((RBLK, D), lambda i: (i, 0)),
        out_shape=jax.ShapeDtypeStruct((Np, D), jnp.float32),
    )(agg2, cnt2, cls16, w1t, b1r, w2t, b2r, gr, ber, srr)


def kernel(x, edge_index, edge_attr, xe1, xe2, ee1, ee2, W1, b1, W2, b2,
           gamma, beta):
    eps = 1e-5
    f32 = jnp.float32
    src = edge_index[0].astype(jnp.int32)
    dst = edge_index[1].astype(jnp.int32)
    pad_e = N + jnp.arange(Ep - E, dtype=jnp.int32) % (Np - N)
    src3 = jnp.concatenate([src, pad_e]).reshape(NW, NCHUNK, ECH)
    dst_p = jnp.concatenate([dst, pad_e])
    dst3 = dst_p.reshape(NW, NCHUNK, ECH)
    c_e = (edge_attr[:, 0] * 3 + edge_attr[:, 1]).astype(jnp.int32)
    i_d = jnp.arange(Ep - E, dtype=jnp.int32)
    pad_c = 16 * (i_d % NW) + 9 + (i_d // NW) % 7 - 16 * (NW - 1)
    c3 = jnp.concatenate([c_e, pad_c]).reshape(NW, NCHUNK, ECH)
    c3 = c3 + (jnp.arange(NW, dtype=jnp.int32) * 16)[:, None, None]
    x01 = (x[:, 0] * 3 + x[:, 1]).astype(jnp.int32)
    x01_3 = jnp.concatenate([x01, jnp.zeros((Np - N,), jnp.int32)]
                            ).reshape(NW, NNCHUNK, NNCH)
    comb = (xe1[:, None, :] + xe2[None, :3, :]).reshape(-1, D)
    oh128 = jnp.tile(jnp.eye(16, D, dtype=f32), (NW, 1))
    z128 = jnp.zeros((Np, D), f32)
    gsc = (gamma / jnp.sqrt(1.0 + eps)).astype(f32)

    h, cnt128 = _precompute(comb, oh128, x01_3, c3, dst3, z128)
    cnt2 = cnt128[:, :, :16]
    for l in range(L):
        cls9 = (ee1[l, :3, None, :] + ee2[l, None, :3, :]).reshape(9, D)
        cls16 = jnp.concatenate([cls9, jnp.zeros((7, D), f32)], 0)
        srr = (ee1[l, 4] + ee2[l, 0]).reshape(1, D)
        agg2 = _agg(h, src3, dst3, z128)
        h = _mlp(l < L - 1, agg2, cnt2, cls16,
                 W1[l].T, b1[l].reshape(1, 2 * D),
                 W2[l].T, b2[l].reshape(1, D),
                 gsc[l].reshape(1, D), beta[l].reshape(1, D), srr)
    return h[:N]


# speedup vs baseline: 15.7046x; 1.1210x over previous
import functools

import jax
import jax.numpy as jnp
from jax import lax
from jax.experimental import pallas as pl
from jax.experimental.pallas import tpu as pltpu
from jax.experimental.pallas import tpu_sc as plsc

N = 10000
E = 320000
D = 128
L = 5

NC = 2
NS = 16
NW = NC * NS
Np = 10240
Ep = NW * Np
EPT = Ep // NW
ECH = 128
NCHUNK = EPT // ECH
NPT = Np // NW
NNCH = 64
NNCHUNK = NPT // NNCH
RPT = Np // NS
HC = NCHUNK // 2
OHR = 4

_mesh = plsc.VectorSubcoreMesh(core_axis_name="c", subcore_axis_name="s")


def _gs_pipeline(src_tab, idx3_hbm, didx3_hbm, w, sidx_v, didx_v,
                 buf0_v, buf1_v, acc_sh, semA, semB):
    for p in range(2):
        pltpu.sync_copy(idx3_hbm.at[w, pl.ds(p * HC, HC)], sidx_v)
        pltpu.sync_copy(didx3_hbm.at[w, pl.ds(p * HC, HC)], didx_v)
        pltpu.async_copy(src_tab.at[sidx_v.at[0]], buf0_v, semA)

        def body(jj, carry):
            j0 = 2 * jj
            d1 = pltpu.async_copy(src_tab.at[sidx_v.at[j0 + 1]], buf1_v, semB)
            pltpu.make_async_copy(src_tab.at[sidx_v.at[j0]], buf0_v,
                                  semA).wait()
            pltpu.sync_copy(buf0_v, acc_sh.at[didx_v.at[j0]], add=True)

            @pl.when(jj < HC // 2 - 1)
            def _():
                pltpu.async_copy(src_tab.at[sidx_v.at[j0 + 2]], buf0_v, semA)

            d1.wait()
            pltpu.sync_copy(buf1_v, acc_sh.at[didx_v.at[j0 + 1]], add=True)
            return carry

        lax.fori_loop(0, HC // 2, body, 0)


def _precompute_body(comb_hbm, oh128_hbm, x01_hbm, c3_hbm, dst3_hbm, z128_hbm,
                     h0_out, cnt_out,
                     nidx_v, cidx_v, didx_v, oh0_v, oh1_v, cnt_sh,
                     semA, semB):
    c = lax.axis_index("c")
    s = lax.axis_index("s")
    w = c * NS + s
    pltpu.sync_copy(x01_hbm.at[w], nidx_v)
    nbuf_v = oh0_v.at[pl.ds(0, NNCH)]
    for j in range(NNCHUNK):
        pltpu.async_copy(comb_hbm.at[nidx_v.at[j]], nbuf_v, semA).wait()
        pltpu.sync_copy(nbuf_v, h0_out.at[pl.ds(w * NPT + j * NNCH, NNCH)])
    rows = pl.ds(s * RPT, RPT)
    pltpu.sync_copy(z128_hbm.at[rows], cnt_sh.at[rows])
    plsc.subcore_barrier()
    _gs_pipeline(oh128_hbm, c3_hbm, dst3_hbm, w, cidx_v, didx_v,
                 oh0_v, oh1_v, cnt_sh, semA, semB)
    plsc.subcore_barrier()
    pltpu.sync_copy(cnt_sh.at[rows], cnt_out.at[c, rows])


_precompute = functools.partial(
    pl.kernel,
    _precompute_body,
    out_type=(
        jax.ShapeDtypeStruct((Np, D), jnp.float32),
        jax.ShapeDtypeStruct((NC, Np, D), jnp.float32),
    ),
    mesh=_mesh,
    scratch_types=[
        pltpu.VMEM((NNCHUNK, NNCH), jnp.int32),
        pltpu.VMEM((HC, ECH), jnp.int32),
        pltpu.VMEM((HC, ECH), jnp.int32),
        pltpu.VMEM((ECH, D), jnp.float32),
        pltpu.VMEM((ECH, D), jnp.float32),
        pltpu.VMEM_SHARED((Np, D), jnp.float32),
        pltpu.SemaphoreType.DMA,
        pltpu.SemaphoreType.DMA,
    ],
)()


def _agg_body(h_hbm, src3_hbm, dst3_hbm, z128_hbm,
              agg_out,
              sidx_v, didx_v, rows0_v, rows1_v, agg_sh, semA, semB):
    c = lax.axis_index("c")
    s = lax.axis_index("s")
    w = c * NS + s
    rows = pl.ds(s * RPT, RPT)

    @pl.when(c == 0)
    def _():
        pltpu.sync_copy(h_hbm.at[rows], agg_sh.at[rows])

    @pl.when(c == 1)
    def _():
        pltpu.sync_copy(z128_hbm.at[rows], agg_sh.at[rows])

    plsc.subcore_barrier()
    _gs_pipeline(h_hbm, src3_hbm, dst3_hbm, w, sidx_v, didx_v,
                 rows0_v, rows1_v, agg_sh, semA, semB)
    plsc.subcore_barrier()
    pltpu.sync_copy(agg_sh.at[rows], agg_out.at[c, rows])


_agg = functools.partial(
    pl.kernel,
    _agg_body,
    out_type=jax.ShapeDtypeStruct((NC, Np, D), jnp.float32),
    mesh=_mesh,
    scratch_types=[
        pltpu.VMEM((HC, ECH), jnp.int32),
        pltpu.VMEM((HC, ECH), jnp.int32),
        pltpu.VMEM((ECH, D), jnp.float32),
        pltpu.VMEM((ECH, D), jnp.float32),
        pltpu.VMEM_SHARED((Np, D), jnp.float32),
        pltpu.SemaphoreType.DMA,
        pltpu.SemaphoreType.DMA,
    ],
)()


RBLK = 512


def _mlp_body(relu, agg_ref, cnt_ref, cls_ref, w1_ref, b1_ref, w2_ref, b2_ref,
              g_ref, be_ref, sr_ref, out_ref):
    z = (agg_ref[0] + agg_ref[1]
         + jnp.dot(cnt_ref[0] + cnt_ref[1], cls_ref[...],
                   preferred_element_type=jnp.float32)
         + sr_ref[...])
    m = jnp.maximum(jnp.dot(z, w1_ref[...],
                            preferred_element_type=jnp.float32) + b1_ref[...],
                    0.0)
    o = jnp.dot(m, w2_ref[...], preferred_element_type=jnp.float32) + b2_ref[...]
    o = o * g_ref[...] + be_ref[...]
    out_ref[...] = jnp.maximum(o, 0.0) if relu else o


def _mlp(relu, agg2, cnt2, cls16, w1t, b1r, w2t, b2r, gr, ber, srr):
    grid = (Np // RBLK,)
    full = lambda shape: pl.BlockSpec(shape, lambda i: (0,) * len(shape))
    return pl.pallas_call(
        functools.partial(_mlp_body, relu),
        grid=grid,
        in_specs=[
            pl.BlockSpec((NC, RBLK, D), lambda i: (0, i, 0)),
            pl.BlockSpec((NC, RBLK, 16), lambda i: (0, i, 0)),
            full((16, D)),
            full((D, 2 * D)),
            full((1, 2 * D)),
            full((2 * D, D)),
            full((1, D)),
            full((1, D)),
            full((1, D)),
            full((1, D)),
        ],
        out_specs=pl.BlockSpec((RBLK, D), lambda i: (i, 0)),
        out_shape=jax.ShapeDtypeStruct((Np, D), jnp.float32),
    )(agg2, cnt2, cls16, w1t, b1r, w2t, b2r, gr, ber, srr)


def kernel(x, edge_index, edge_attr, xe1, xe2, ee1, ee2, W1, b1, W2, b2,
           gamma, beta):
    eps = 1e-5
    f32 = jnp.float32
    src = edge_index[0].astype(jnp.int32)
    dst = edge_index[1].astype(jnp.int32)
    pad_e = N + jnp.arange(Ep - E, dtype=jnp.int32) % (Np - N)
    src3 = jnp.concatenate([src, pad_e]).reshape(NW, NCHUNK, ECH)
    dst_p = jnp.concatenate([dst, pad_e])
    dst3 = dst_p.reshape(NW, NCHUNK, ECH)
    c_e = (edge_attr[:, 0] * 3 + edge_attr[:, 1]).astype(jnp.int32)
    i_d = jnp.arange(Ep - E, dtype=jnp.int32)
    pad_c = (16 * OHR * (i_d % NW) + 9 + (i_d // NW) % 7
             - 16 * OHR * (NW - 1))
    c3 = jnp.concatenate([c_e, pad_c]).reshape(NW, NCHUNK, ECH)
    c3 = (c3 + (jnp.arange(NW, dtype=jnp.int32) * 16 * OHR)[:, None, None]
          + (jnp.arange(NCHUNK, dtype=jnp.int32) % OHR * 16)[None, :, None])
    x01 = (x[:, 0] * 3 + x[:, 1]).astype(jnp.int32)
    x01 = x01 + jnp.repeat(jnp.arange(NW, dtype=jnp.int32) * 360, NPT)[:N]
    pad_n = (NW - 1) * 360 + jnp.arange(Np - N, dtype=jnp.int32) % 360
    x01_3 = jnp.concatenate([x01, pad_n]).reshape(NW, NNCHUNK, NNCH)
    comb = jnp.tile((xe1[:, None, :] + xe2[None, :3, :]).reshape(-1, D),
                    (NW, 1))
    oh128 = jnp.tile(jnp.eye(16, D, dtype=f32), (NW * OHR, 1))
    z128 = jnp.zeros((Np, D), f32)
    gsc = (gamma / jnp.sqrt(1.0 + eps)).astype(f32)

    h, cnt128 = _precompute(comb, oh128, x01_3, c3, dst3, z128)
    cnt2 = cnt128[:, :, :16]
    for l in range(L):
        cls9 = (ee1[l, :3, None, :] + ee2[l, None, :3, :]).reshape(9, D)
        cls16 = jnp.concatenate([cls9, jnp.zeros((7, D), f32)], 0)
        srr = (ee1[l, 4] + ee2[l, 0]).reshape(1, D)
        agg2 = _agg(h, src3, dst3, z128)
        h = _mlp(l < L - 1, agg2, cnt2, cls16,
                 W1[l].T, b1[l].reshape(1, 2 * D),
                 W2[l].T, b2[l].reshape(1, D),
                 gsc[l].reshape(1, D), beta[l].reshape(1, D), srr)
    return h[:N]


# speedup vs baseline: 16.5571x; 1.0543x over previous
import functools

import jax
import jax.numpy as jnp
from jax import lax
from jax.experimental import pallas as pl
from jax.experimental.pallas import tpu as pltpu
from jax.experimental.pallas import tpu_sc as plsc

N = 10000
E = 320000
D = 128
L = 5

NC = 2
NS = 16
NW = NC * NS
Np = 10240
Ep = NW * Np
EPT = Ep // NW
ECH = 64
NCHUNK = EPT // ECH
NPT = Np // NW
NNCH = 64
NNCHUNK = NPT // NNCH
RPT = Np // NS
HC = NCHUNK // 2
NBUF = 3
OHR = 4

_mesh = plsc.VectorSubcoreMesh(core_axis_name="c", subcore_axis_name="s")


def _gs_pipeline(src_tab, idx3_hbm, didx3_hbm, w, sidx_v, didx_v,
                 bufs, acc_sh, sems):
    for p in range(2):
        pltpu.sync_copy(idx3_hbm.at[w, pl.ds(p * HC, HC)], sidx_v)
        pltpu.sync_copy(didx3_hbm.at[w, pl.ds(p * HC, HC)], didx_v)
        for k in range(NBUF - 1):
            pltpu.async_copy(src_tab.at[sidx_v.at[k]], bufs[k], sems[k])

        def body(jj, carry):
            for k in range(NBUF):
                j = NBUF * jj + k
                ka = (k + NBUF - 1) % NBUF

                @pl.when(j + NBUF - 1 < HC)
                def _():
                    pltpu.async_copy(src_tab.at[sidx_v.at[j + NBUF - 1]],
                                     bufs[ka], sems[ka])

                pltpu.make_async_copy(src_tab.at[sidx_v.at[j]], bufs[k],
                                      sems[k]).wait()
                pltpu.sync_copy(bufs[k], acc_sh.at[didx_v.at[j]], add=True)
            return carry

        lax.fori_loop(0, HC // NBUF, body, 0)
        for r in range(HC - HC % NBUF, HC):
            pltpu.make_async_copy(src_tab.at[sidx_v.at[r]], bufs[r % NBUF],
                                  sems[r % NBUF]).wait()
            pltpu.sync_copy(bufs[r % NBUF], acc_sh.at[didx_v.at[r]], add=True)


def _precompute_body(comb_hbm, oh128_hbm, x01_hbm, c3_hbm, dst3_hbm, z128_hbm,
                     h0_out, cnt_out,
                     nidx_v, cidx_v, didx_v, b0, b1, b2, cnt_sh,
                     s0, s1, s2):
    bufs, sems = [b0, b1, b2], [s0, s1, s2]
    c = lax.axis_index("c")
    s = lax.axis_index("s")
    w = c * NS + s
    pltpu.sync_copy(x01_hbm.at[w], nidx_v)
    for j in range(min(NBUF, NNCHUNK)):
        pltpu.async_copy(comb_hbm.at[nidx_v.at[j]], bufs[j % NBUF],
                         sems[j % NBUF])
    for j in range(NNCHUNK):
        pltpu.make_async_copy(comb_hbm.at[nidx_v.at[j]], bufs[j % NBUF],
                              sems[j % NBUF]).wait()
        pltpu.sync_copy(bufs[j % NBUF],
                        h0_out.at[pl.ds(w * NPT + j * NNCH, NNCH)])
        if j + NBUF < NNCHUNK:
            pltpu.async_copy(comb_hbm.at[nidx_v.at[j + NBUF]],
                             bufs[j % NBUF], sems[j % NBUF])
    rows = pl.ds(s * RPT, RPT)
    pltpu.sync_copy(z128_hbm.at[rows], cnt_sh.at[rows])
    plsc.subcore_barrier()
    _gs_pipeline(oh128_hbm, c3_hbm, dst3_hbm, w, cidx_v, didx_v,
                 bufs, cnt_sh, sems)
    plsc.subcore_barrier()
    pltpu.sync_copy(cnt_sh.at[rows], cnt_out.at[c, rows])


_precompute = functools.partial(
    pl.kernel,
    _precompute_body,
    out_type=(
        jax.ShapeDtypeStruct((Np, D), jnp.float32),
        jax.ShapeDtypeStruct((NC, Np, D), jnp.float32),
    ),
    mesh=_mesh,
    scratch_types=[
        pltpu.VMEM((NNCHUNK, NNCH), jnp.int32),
        pltpu.VMEM((HC, ECH), jnp.int32),
        pltpu.VMEM((HC, ECH), jnp.int32),
        pltpu.VMEM((ECH, D), jnp.float32),
        pltpu.VMEM((ECH, D), jnp.float32),
        pltpu.VMEM((ECH, D), jnp.float32),
        pltpu.VMEM_SHARED((Np, D), jnp.float32),
        pltpu.SemaphoreType.DMA,
        pltpu.SemaphoreType.DMA,
        pltpu.SemaphoreType.DMA,
    ],
)()


def _agg_body(h_hbm, src3_hbm, dst3_hbm, z128_hbm,
              agg_out,
              sidx_v, didx_v, b0, b1, b2, agg_sh, s0, s1, s2):
    bufs, sems = [b0, b1, b2], [s0, s1, s2]
    c = lax.axis_index("c")
    s = lax.axis_index("s")
    w = c * NS + s
    rows = pl.ds(s * RPT, RPT)

    @pl.when(c == 0)
    def _():
        pltpu.sync_copy(h_hbm.at[rows], agg_sh.at[rows])

    @pl.when(c == 1)
    def _():
        pltpu.sync_copy(z128_hbm.at[rows], agg_sh.at[rows])

    plsc.subcore_barrier()
    _gs_pipeline(h_hbm, src3_hbm, dst3_hbm, w, sidx_v, didx_v,
                 bufs, agg_sh, sems)
    plsc.subcore_barrier()
    pltpu.sync_copy(agg_sh.at[rows], agg_out.at[c, rows])


_agg = functools.partial(
    pl.kernel,
    _agg_body,
    out_type=jax.ShapeDtypeStruct((NC, Np, D), jnp.float32),
    mesh=_mesh,
    scratch_types=[
        pltpu.VMEM((HC, ECH), jnp.int32),
        pltpu.VMEM((HC, ECH), jnp.int32),
        pltpu.VMEM((ECH, D), jnp.float32),
        pltpu.VMEM((ECH, D), jnp.float32),
        pltpu.VMEM((ECH, D), jnp.float32),
        pltpu.VMEM_SHARED((Np, D), jnp.float32),
        pltpu.SemaphoreType.DMA,
        pltpu.SemaphoreType.DMA,
        pltpu.SemaphoreType.DMA,
    ],
)()


RBLK = 512


def _mlp_body(relu, agg_ref, cnt_ref, cls_ref, w1_ref, b1_ref, w2_ref, b2_ref,
              g_ref, be_ref, sr_ref, out_ref):
    z = (agg_ref[0] + agg_ref[1]
         + jnp.dot(cnt_ref[0] + cnt_ref[1], cls_ref[...],
                   preferred_element_type=jnp.float32)
         + sr_ref[...])
    m = jnp.maximum(jnp.dot(z, w1_ref[...],
                            preferred_element_type=jnp.float32) + b1_ref[...],
                    0.0)
    o = jnp.dot(m, w2_ref[...], preferred_element_type=jnp.float32) + b2_ref[...]
    o = o * g_ref[...] + be_ref[...]
    out_ref[...] = jnp.maximum(o, 0.0) if relu else o


def _mlp(relu, agg2, cnt2, cls16, w1t, b1r, w2t, b2r, gr, ber, srr):
    grid = (Np // RBLK,)
    full = lambda shape: pl.BlockSpec(shape, lambda i: (0,) * len(shape))
    return pl.pallas_call(
        functools.partial(_mlp_body, relu),
        grid=grid,
        in_specs=[
            pl.BlockSpec((NC, RBLK, D), lambda i: (0, i, 0)),
            pl.BlockSpec((NC, RBLK, 16), lambda i: (0, i, 0)),
            full((16, D)),
            full((D, 2 * D)),
            full((1, 2 * D)),
            full((2 * D, D)),
            full((1, D)),
            full((1, D)),
            full((1, D)),
            full((1, D)),
        ],
        out_specs=pl.BlockSpec((RBLK, D), lambda i: (i, 0)),
        out_shape=jax.ShapeDtypeStruct((Np, D), jnp.float32),
    )(agg2, cnt2, cls16, w1t, b1r, w2t, b2r, gr, ber, srr)


def kernel(x, edge_index, edge_attr, xe1, xe2, ee1, ee2, W1, b1, W2, b2,
           gamma, beta):
    eps = 1e-5
    f32 = jnp.float32
    src = edge_index[0].astype(jnp.int32)
    dst = edge_index[1].astype(jnp.int32)
    pad_e = N + jnp.arange(Ep - E, dtype=jnp.int32) % (Np - N)
    src3 = jnp.concatenate([src, pad_e]).reshape(NW, NCHUNK, ECH)
    dst_p = jnp.concatenate([dst, pad_e])
    dst3 = dst_p.reshape(NW, NCHUNK, ECH)
    c_e = (edge_attr[:, 0] * 3 + edge_attr[:, 1]).astype(jnp.int32)
    i_d = jnp.arange(Ep - E, dtype=jnp.int32)
    pad_c = (16 * OHR * (i_d % NW) + 9 + (i_d // NW) % 7
             - 16 * OHR * (NW - 1))
    c3 = jnp.concatenate([c_e, pad_c]).reshape(NW, NCHUNK, ECH)
    c3 = (c3 + (jnp.arange(NW, dtype=jnp.int32) * 16 * OHR)[:, None, None]
          + (jnp.arange(NCHUNK, dtype=jnp.int32) % OHR * 16)[None, :, None])
    x01 = (x[:, 0] * 3 + x[:, 1]).astype(jnp.int32)
    x01 = x01 + jnp.repeat(jnp.arange(NW, dtype=jnp.int32) * 360, NPT)[:N]
    pad_n = (NW - 1) * 360 + jnp.arange(Np - N, dtype=jnp.int32) % 360
    x01_3 = jnp.concatenate([x01, pad_n]).reshape(NW, NNCHUNK, NNCH)
    comb = jnp.tile((xe1[:, None, :] + xe2[None, :3, :]).reshape(-1, D),
                    (NW, 1))
    oh128 = jnp.tile(jnp.eye(16, D, dtype=f32), (NW * OHR, 1))
    z128 = jnp.zeros((Np, D), f32)
    gsc = (gamma / jnp.sqrt(1.0 + eps)).astype(f32)

    h, cnt128 = _precompute(comb, oh128, x01_3, c3, dst3, z128)
    cnt2 = cnt128[:, :, :16]
    for l in range(L):
        cls9 = (ee1[l, :3, None, :] + ee2[l, None, :3, :]).reshape(9, D)
        cls16 = jnp.concatenate([cls9, jnp.zeros((7, D), f32)], 0)
        srr = (ee1[l, 4] + ee2[l, 0]).reshape(1, D)
        agg2 = _agg(h, src3, dst3, z128)
        h = _mlp(l < L - 1, agg2, cnt2, cls16,
                 W1[l].T, b1[l].reshape(1, 2 * D),
                 W2[l].T, b2[l].reshape(1, D),
                 gsc[l].reshape(1, D), beta[l].reshape(1, D), srr)
    return h[:N]


# speedup vs baseline: 16.6992x; 1.0086x over previous
import functools

import jax
import jax.numpy as jnp
from jax import lax
from jax.experimental import pallas as pl
from jax.experimental.pallas import tpu as pltpu
from jax.experimental.pallas import tpu_sc as plsc

N = 10000
E = 320000
D = 128
L = 5

NC = 2
NS = 16
NW = NC * NS
Np = 10240
Ep = NW * Np
EPT = Ep // NW
ECH = 64
NCHUNK = EPT // ECH
NPT = Np // NW
NNCH = 64
NNCHUNK = NPT // NNCH
RPT = Np // NS
HC = NCHUNK // 2
NBUF = 3
OHR = 4

_mesh = plsc.VectorSubcoreMesh(core_axis_name="c", subcore_axis_name="s")


def _gs_pipeline(src_tab, idx3_hbm, didx3_hbm, w, sidx_v, didx_v,
                 bufs, acc_sh, sems):
    for p in range(2):
        pltpu.sync_copy(idx3_hbm.at[w, pl.ds(p * HC, HC)], sidx_v)
        pltpu.sync_copy(didx3_hbm.at[w, pl.ds(p * HC, HC)], didx_v)
        for k in range(NBUF - 1):
            pltpu.async_copy(src_tab.at[sidx_v.at[k]], bufs[k], sems[k])

        def body(jj, carry):
            for k in range(NBUF):
                j = NBUF * jj + k
                ka = (k + NBUF - 1) % NBUF

                @pl.when(j + NBUF - 1 < HC)
                def _():
                    pltpu.async_copy(src_tab.at[sidx_v.at[j + NBUF - 1]],
                                     bufs[ka], sems[ka])

                pltpu.make_async_copy(src_tab.at[sidx_v.at[j]], bufs[k],
                                      sems[k]).wait()
                pltpu.sync_copy(bufs[k], acc_sh.at[didx_v.at[j]], add=True)
            return carry

        lax.fori_loop(0, HC // NBUF, body, 0)
        for r in range(HC - HC % NBUF, HC):
            pltpu.make_async_copy(src_tab.at[sidx_v.at[r]], bufs[r % NBUF],
                                  sems[r % NBUF]).wait()
            pltpu.sync_copy(bufs[r % NBUF], acc_sh.at[didx_v.at[r]], add=True)


def _count_body(oh128_hbm, c3_hbm, dst3_hbm, z128_hbm,
                cnt_out,
                cidx_v, didx_v, b0, b1, b2, cnt_sh,
                s0, s1, s2):
    bufs, sems = [b0, b1, b2], [s0, s1, s2]
    c = lax.axis_index("c")
    s = lax.axis_index("s")
    w = c * NS + s
    rows = pl.ds(s * RPT, RPT)
    pltpu.sync_copy(z128_hbm.at[rows], cnt_sh.at[rows])
    plsc.subcore_barrier()
    _gs_pipeline(oh128_hbm, c3_hbm, dst3_hbm, w, cidx_v, didx_v,
                 bufs, cnt_sh, sems)
    plsc.subcore_barrier()
    pltpu.sync_copy(cnt_sh.at[rows], cnt_out.at[c, rows])


_count = functools.partial(
    pl.kernel,
    _count_body,
    out_type=jax.ShapeDtypeStruct((NC, Np, D), jnp.float32),
    mesh=_mesh,
    scratch_types=[
        pltpu.VMEM((HC, ECH), jnp.int32),
        pltpu.VMEM((HC, ECH), jnp.int32),
        pltpu.VMEM((ECH, D), jnp.float32),
        pltpu.VMEM((ECH, D), jnp.float32),
        pltpu.VMEM((ECH, D), jnp.float32),
        pltpu.VMEM_SHARED((Np, D), jnp.float32),
        pltpu.SemaphoreType.DMA,
        pltpu.SemaphoreType.DMA,
        pltpu.SemaphoreType.DMA,
    ],
)()


def _agg_body(h_hbm, src3_hbm, dst3_hbm, z128_hbm,
              agg_out,
              sidx_v, didx_v, b0, b1, b2, agg_sh, s0, s1, s2):
    bufs, sems = [b0, b1, b2], [s0, s1, s2]
    c = lax.axis_index("c")
    s = lax.axis_index("s")
    w = c * NS + s
    rows = pl.ds(s * RPT, RPT)

    @pl.when(c == 0)
    def _():
        pltpu.sync_copy(h_hbm.at[rows], agg_sh.at[rows])

    @pl.when(c == 1)
    def _():
        pltpu.sync_copy(z128_hbm.at[rows], agg_sh.at[rows])

    plsc.subcore_barrier()
    _gs_pipeline(h_hbm, src3_hbm, dst3_hbm, w, sidx_v, didx_v,
                 bufs, agg_sh, sems)
    plsc.subcore_barrier()
    pltpu.sync_copy(agg_sh.at[rows], agg_out.at[c, rows])


_agg = functools.partial(
    pl.kernel,
    _agg_body,
    out_type=jax.ShapeDtypeStruct((NC, Np, D), jnp.float32),
    mesh=_mesh,
    scratch_types=[
        pltpu.VMEM((HC, ECH), jnp.int32),
        pltpu.VMEM((HC, ECH), jnp.int32),
        pltpu.VMEM((ECH, D), jnp.float32),
        pltpu.VMEM((ECH, D), jnp.float32),
        pltpu.VMEM((ECH, D), jnp.float32),
        pltpu.VMEM_SHARED((Np, D), jnp.float32),
        pltpu.SemaphoreType.DMA,
        pltpu.SemaphoreType.DMA,
        pltpu.SemaphoreType.DMA,
    ],
)()


RBLK = 512


def _h0_body(idx_ref, comb_ref, out_ref):
    idx = idx_ref[...]
    onehot = jnp.where(idx == lax.broadcasted_iota(jnp.int32, (RBLK, 384), 1),
                       1.0, 0.0)
    out_ref[...] = jnp.dot(onehot, comb_ref[...],
                           preferred_element_type=jnp.float32)


def _h0_tc(x01p, comb384):
    return pl.pallas_call(
        _h0_body,
        grid=(Np // RBLK,),
        in_specs=[
            pl.BlockSpec((RBLK, 1), lambda i: (i, 0)),
            pl.BlockSpec((384, D), lambda i: (0, 0)),
        ],
        out_specs=pl.BlockSpec((RBLK, D), lambda i: (i, 0)),
        out_shape=jax.ShapeDtypeStruct((Np, D), jnp.float32),
    )(x01p, comb384)


def _mlp_body(relu, agg_ref, cnt_ref, cls_ref, w1_ref, b1_ref, w2_ref, b2_ref,
              g_ref, be_ref, sr_ref, out_ref):
    z = (agg_ref[0] + agg_ref[1]
         + jnp.dot(cnt_ref[0] + cnt_ref[1], cls_ref[...],
                   preferred_element_type=jnp.float32)
         + sr_ref[...])
    m = jnp.maximum(jnp.dot(z, w1_ref[...],
                            preferred_element_type=jnp.float32) + b1_ref[...],
                    0.0)
    o = jnp.dot(m, w2_ref[...], preferred_element_type=jnp.float32) + b2_ref[...]
    o = o * g_ref[...] + be_ref[...]
    out_ref[...] = jnp.maximum(o, 0.0) if relu else o


def _mlp(relu, agg2, cnt2, cls16, w1t, b1r, w2t, b2r, gr, ber, srr):
    grid = (Np // RBLK,)
    full = lambda shape: pl.BlockSpec(shape, lambda i: (0,) * len(shape))
    return pl.pallas_call(
        functools.partial(_mlp_body, relu),
        grid=grid,
        in_specs=[
            pl.BlockSpec((NC, RBLK, D), lambda i: (0, i, 0)),
            pl.BlockSpec((NC, RBLK, 16), lambda i: (0, i, 0)),
            full((16, D)),
            full((D, 2 * D)),
            full((1, 2 * D)),
            full((2 * D, D)),
            full((1, D)),
            full((1, D)),
            full((1, D)),
            full((1, D)),
        ],
        out_specs=pl.BlockSpec((RBLK, D), lambda i: (i, 0)),
        out_shape=jax.ShapeDtypeStruct((Np, D), jnp.float32),
    )(agg2, cnt2, cls16, w1t, b1r, w2t, b2r, gr, ber, srr)


def kernel(x, edge_index, edge_attr, xe1, xe2, ee1, ee2, W1, b1, W2, b2,
           gamma, beta):
    eps = 1e-5
    f32 = jnp.float32
    src = edge_index[0].astype(jnp.int32)
    dst = edge_index[1].astype(jnp.int32)
    pad_e = N + jnp.arange(Ep - E, dtype=jnp.int32) % (Np - N)
    src3 = jnp.concatenate([src, pad_e]).reshape(NW, NCHUNK, ECH)
    dst_p = jnp.concatenate([dst, pad_e])
    dst3 = dst_p.reshape(NW, NCHUNK, ECH)
    c_e = (edge_attr[:, 0] * 3 + edge_attr[:, 1]).astype(jnp.int32)
    i_d = jnp.arange(Ep - E, dtype=jnp.int32)
    pad_c = (16 * OHR * (i_d % NW) + 9 + (i_d // NW) % 7
             - 16 * OHR * (NW - 1))
    c3 = jnp.concatenate([c_e, pad_c]).reshape(NW, NCHUNK, ECH)
    c3 = (c3 + (jnp.arange(NW, dtype=jnp.int32) * 16 * OHR)[:, None, None]
          + (jnp.arange(NCHUNK, dtype=jnp.int32) % OHR * 16)[None, :, None])
    x01 = (x[:, 0] * 3 + x[:, 1]).astype(jnp.int32)
    x01p = jnp.concatenate([x01, jnp.zeros((Np - N,), jnp.int32)]
                           ).reshape(Np, 1)
    comb384 = jnp.concatenate(
        [(xe1[:, None, :] + xe2[None, :3, :]).reshape(-1, D),
         jnp.zeros((24, D), f32)], 0)
    oh128 = jnp.tile(jnp.eye(16, D, dtype=f32), (NW * OHR, 1))
    z128 = jnp.zeros((Np, D), f32)
    gsc = (gamma / jnp.sqrt(1.0 + eps)).astype(f32)

    cnt128 = _count(oh128, c3, dst3, z128)
    cnt2 = cnt128[:, :, :16]
    h = _h0_tc(x01p, comb384)
    for l in range(L):
        cls9 = (ee1[l, :3, None, :] + ee2[l, None, :3, :]).reshape(9, D)
        cls16 = jnp.concatenate([cls9, jnp.zeros((7, D), f32)], 0)
        srr = (ee1[l, 4] + ee2[l, 0]).reshape(1, D)
        agg2 = _agg(h, src3, dst3, z128)
        h = _mlp(l < L - 1, agg2, cnt2, cls16,
                 W1[l].T, b1[l].reshape(1, 2 * D),
                 W2[l].T, b2[l].reshape(1, D),
                 gsc[l].reshape(1, D), beta[l].reshape(1, D), srr)
    return h[:N]


# speedup vs baseline: 16.7845x; 1.0051x over previous
import functools

import jax
import jax.numpy as jnp
from jax import lax
from jax.experimental import pallas as pl
from jax.experimental.pallas import tpu as pltpu
from jax.experimental.pallas import tpu_sc as plsc

N = 10000
E = 320000
D = 128
L = 5

NC = 2
NS = 16
NW = NC * NS
Np = 10240
Ep = NW * Np
EPT = Ep // NW
ECH = 64
NCHUNK = EPT // ECH
NPT = Np // NW
NNCH = 64
NNCHUNK = NPT // NNCH
RPT = Np // NS
HC = NCHUNK // 2
NBUF = 3
OHR = 16

_mesh = plsc.VectorSubcoreMesh(core_axis_name="c", subcore_axis_name="s")


def _gs_pipeline(src_tab, idx3_hbm, didx3_hbm, w, sidx_v, didx_v,
                 bufs, acc_sh, sems):
    for p in range(2):
        pltpu.sync_copy(idx3_hbm.at[w, pl.ds(p * HC, HC)], sidx_v)
        pltpu.sync_copy(didx3_hbm.at[w, pl.ds(p * HC, HC)], didx_v)
        for k in range(NBUF - 1):
            pltpu.async_copy(src_tab.at[sidx_v.at[k]], bufs[k], sems[k])

        def body(jj, carry):
            for k in range(NBUF):
                j = NBUF * jj + k
                ka = (k + NBUF - 1) % NBUF

                @pl.when(j + NBUF - 1 < HC)
                def _():
                    pltpu.async_copy(src_tab.at[sidx_v.at[j + NBUF - 1]],
                                     bufs[ka], sems[ka])

                pltpu.make_async_copy(src_tab.at[sidx_v.at[j]], bufs[k],
                                      sems[k]).wait()
                pltpu.sync_copy(bufs[k], acc_sh.at[didx_v.at[j]], add=True)
            return carry

        lax.fori_loop(0, HC // NBUF, body, 0)
        for r in range(HC - HC % NBUF, HC):
            pltpu.make_async_copy(src_tab.at[sidx_v.at[r]], bufs[r % NBUF],
                                  sems[r % NBUF]).wait()
            pltpu.sync_copy(bufs[r % NBUF], acc_sh.at[didx_v.at[r]], add=True)


def _count_body(oh128_hbm, c3_hbm, dst3_hbm, z128_hbm,
                cnt_out,
                cidx_v, didx_v, b0, b1, b2, cnt_sh,
                s0, s1, s2):
    bufs, sems = [b0, b1, b2], [s0, s1, s2]
    c = lax.axis_index("c")
    s = lax.axis_index("s")
    w = c * NS + s
    rows = pl.ds(s * RPT, RPT)
    pltpu.sync_copy(z128_hbm.at[rows], cnt_sh.at[rows])
    plsc.subcore_barrier()
    _gs_pipeline(oh128_hbm, c3_hbm, dst3_hbm, w, cidx_v, didx_v,
                 bufs, cnt_sh, sems)
    plsc.subcore_barrier()
    pltpu.sync_copy(cnt_sh.at[rows], cnt_out.at[c, rows])


_count = functools.partial(
    pl.kernel,
    _count_body,
    out_type=jax.ShapeDtypeStruct((NC, Np, D), jnp.float32),
    mesh=_mesh,
    scratch_types=[
        pltpu.VMEM((HC, ECH), jnp.int32),
        pltpu.VMEM((HC, ECH), jnp.int32),
        pltpu.VMEM((ECH, D), jnp.float32),
        pltpu.VMEM((ECH, D), jnp.float32),
        pltpu.VMEM((ECH, D), jnp.float32),
        pltpu.VMEM_SHARED((Np, D), jnp.float32),
        pltpu.SemaphoreType.DMA,
        pltpu.SemaphoreType.DMA,
        pltpu.SemaphoreType.DMA,
    ],
)()


def _agg_body(h_hbm, src3_hbm, dst3_hbm, z128_hbm,
              agg_out,
              sidx_v, didx_v, b0, b1, b2, agg_sh, s0, s1, s2):
    bufs, sems = [b0, b1, b2], [s0, s1, s2]
    c = lax.axis_index("c")
    s = lax.axis_index("s")
    w = c * NS + s
    rows = pl.ds(s * RPT, RPT)

    @pl.when(c == 0)
    def _():
        pltpu.sync_copy(h_hbm.at[rows], agg_sh.at[rows])

    @pl.when(c == 1)
    def _():
        pltpu.sync_copy(z128_hbm.at[rows], agg_sh.at[rows])

    plsc.subcore_barrier()
    _gs_pipeline(h_hbm, src3_hbm, dst3_hbm, w, sidx_v, didx_v,
                 bufs, agg_sh, sems)
    plsc.subcore_barrier()
    pltpu.sync_copy(agg_sh.at[rows], agg_out.at[c, rows])


_agg = functools.partial(
    pl.kernel,
    _agg_body,
    out_type=jax.ShapeDtypeStruct((NC, Np, D), jnp.float32),
    mesh=_mesh,
    scratch_types=[
        pltpu.VMEM((HC, ECH), jnp.int32),
        pltpu.VMEM((HC, ECH), jnp.int32),
        pltpu.VMEM((ECH, D), jnp.float32),
        pltpu.VMEM((ECH, D), jnp.float32),
        pltpu.VMEM((ECH, D), jnp.float32),
        pltpu.VMEM_SHARED((Np, D), jnp.float32),
        pltpu.SemaphoreType.DMA,
        pltpu.SemaphoreType.DMA,
        pltpu.SemaphoreType.DMA,
    ],
)()


RBLK = 512


def _h0_body(idx_ref, comb_ref, out_ref):
    idx = idx_ref[...]
    onehot = jnp.where(idx == lax.broadcasted_iota(jnp.int32, (RBLK, 384), 1),
                       1.0, 0.0)
    out_ref[...] = jnp.dot(onehot, comb_ref[...],
                           preferred_element_type=jnp.float32)


def _h0_tc(x01p, comb384):
    return pl.pallas_call(
        _h0_body,
        grid=(Np // RBLK,),
        in_specs=[
            pl.BlockSpec((RBLK, 1), lambda i: (i, 0)),
            pl.BlockSpec((384, D), lambda i: (0, 0)),
        ],
        out_specs=pl.BlockSpec((RBLK, D), lambda i: (i, 0)),
        out_shape=jax.ShapeDtypeStruct((Np, D), jnp.float32),
    )(x01p, comb384)


def _mlp_body(relu, agg_ref, cnt_ref, cls_ref, w1_ref, b1_ref, w2_ref, b2_ref,
              g_ref, be_ref, sr_ref, out_ref):
    z = (agg_ref[0] + agg_ref[1]
         + jnp.dot(cnt_ref[0] + cnt_ref[1], cls_ref[...],
                   preferred_element_type=jnp.float32)
         + sr_ref[...])
    m = jnp.maximum(jnp.dot(z, w1_ref[...],
                            preferred_element_type=jnp.float32) + b1_ref[...],
                    0.0)
    o = jnp.dot(m, w2_ref[...], preferred_element_type=jnp.float32) + b2_ref[...]
    o = o * g_ref[...] + be_ref[...]
    out_ref[...] = jnp.maximum(o, 0.0) if relu else o


def _mlp(relu, agg2, cnt2, cls16, w1t, b1r, w2t, b2r, gr, ber, srr):
    grid = (Np // RBLK,)
    full = lambda shape: pl.BlockSpec(shape, lambda i: (0,) * len(shape))
    return pl.pallas_call(
        functools.partial(_mlp_body, relu),
        grid=grid,
        in_specs=[
            pl.BlockSpec((NC, RBLK, D), lambda i: (0, i, 0)),
            pl.BlockSpec((NC, RBLK, 16), lambda i: (0, i, 0)),
            full((16, D)),
            full((D, 2 * D)),
            full((1, 2 * D)),
            full((2 * D, D)),
            full((1, D)),
            full((1, D)),
            full((1, D)),
            full((1, D)),
        ],
        out_specs=pl.BlockSpec((RBLK, D), lambda i: (i, 0)),
        out_shape=jax.ShapeDtypeStruct((Np, D), jnp.float32),
    )(agg2, cnt2, cls16, w1t, b1r, w2t, b2r, gr, ber, srr)


def kernel(x, edge_index, edge_attr, xe1, xe2, ee1, ee2, W1, b1, W2, b2,
           gamma, beta):
    eps = 1e-5
    f32 = jnp.float32
    src = edge_index[0].astype(jnp.int32)
    dst = edge_index[1].astype(jnp.int32)
    pad_e = N + jnp.arange(Ep - E, dtype=jnp.int32) % (Np - N)
    src3 = jnp.concatenate([src, pad_e]).reshape(NW, NCHUNK, ECH)
    dst_p = jnp.concatenate([dst, pad_e])
    dst3 = dst_p.reshape(NW, NCHUNK, ECH)
    c_e = (edge_attr[:, 0] * 3 + edge_attr[:, 1]).astype(jnp.int32)
    i_d = jnp.arange(Ep - E, dtype=jnp.int32)
    pad_c = (16 * OHR * (i_d % NW) + 9 + (i_d // NW) % 7
             - 16 * OHR * (NW - 1))
    c3 = jnp.concatenate([c_e, pad_c]).reshape(NW, NCHUNK, ECH)
    c3 = (c3 + (jnp.arange(NW, dtype=jnp.int32) * 16 * OHR)[:, None, None]
          + (jnp.arange(NCHUNK, dtype=jnp.int32) % OHR * 16)[None, :, None])
    x01 = (x[:, 0] * 3 + x[:, 1]).astype(jnp.int32)
    x01p = jnp.concatenate([x01, jnp.zeros((Np - N,), jnp.int32)]
                           ).reshape(Np, 1)
    comb384 = jnp.concatenate(
        [(xe1[:, None, :] + xe2[None, :3, :]).reshape(-1, D),
         jnp.zeros((24, D), f32)], 0)
    oh128 = jnp.tile(jnp.eye(16, D, dtype=f32), (NW * OHR, 1))
    z128 = jnp.zeros((Np, D), f32)
    gsc = (gamma / jnp.sqrt(1.0 + eps)).astype(f32)

    cnt128 = _count(oh128, c3, dst3, z128)
    cnt2 = cnt128[:, :, :16]
    h = _h0_tc(x01p, comb384)
    for l in range(L):
        cls9 = (ee1[l, :3, None, :] + ee2[l, None, :3, :]).reshape(9, D)
        cls16 = jnp.concatenate([cls9, jnp.zeros((7, D), f32)], 0)
        srr = (ee1[l, 4] + ee2[l, 0]).reshape(1, D)
        agg2 = _agg(h, src3, dst3, z128)
        h = _mlp(l < L - 1, agg2, cnt2, cls16,
                 W1[l].T, b1[l].reshape(1, 2 * D),
                 W2[l].T, b2[l].reshape(1, D),
                 gsc[l].reshape(1, D), beta[l].reshape(1, D), srr)
    return h[:N]


# speedup vs baseline: 17.4533x; 1.0398x over previous
import functools

import jax
import jax.numpy as jnp
from jax import lax
from jax.experimental import pallas as pl
from jax.experimental.pallas import tpu as pltpu
from jax.experimental.pallas import tpu_sc as plsc

N = 10000
E = 320000
D = 128
L = 5

NC = 2
NS = 16
NW = NC * NS
Np = 10240
Ep = NW * Np
EPT = Ep // NW
ECH = 64
NCHUNK = EPT // ECH
NPT = Np // NW
NNCH = 64
NNCHUNK = NPT // NNCH
RPT = Np // NS
HC = NCHUNK // 2
NBUF = 3
OHR = 16

_mesh = plsc.VectorSubcoreMesh(core_axis_name="c", subcore_axis_name="s")


def _gs_pipeline(src_tab, idx3_hbm, didx3_hbm, w, sidx_v, didx_v,
                 bufs, acc_sh, sems):
    for p in range(2):
        pltpu.sync_copy(idx3_hbm.at[w, pl.ds(p * HC, HC)], sidx_v)
        pltpu.sync_copy(didx3_hbm.at[w, pl.ds(p * HC, HC)], didx_v)
        for k in range(NBUF - 1):
            pltpu.async_copy(src_tab.at[sidx_v.at[k]], bufs[k], sems[k])

        def body(jj, carry):
            for k in range(NBUF):
                j = NBUF * jj + k
                ka = (k + NBUF - 1) % NBUF

                @pl.when(j + NBUF - 1 < HC)
                def _():
                    pltpu.async_copy(src_tab.at[sidx_v.at[j + NBUF - 1]],
                                     bufs[ka], sems[ka])

                pltpu.make_async_copy(src_tab.at[sidx_v.at[j]], bufs[k],
                                      sems[k]).wait()
                pltpu.sync_copy(bufs[k], acc_sh.at[didx_v.at[j]], add=True)
            return carry

        lax.fori_loop(0, HC // NBUF, body, 0)
        for r in range(HC - HC % NBUF, HC):
            pltpu.make_async_copy(src_tab.at[sidx_v.at[r]], bufs[r % NBUF],
                                  sems[r % NBUF]).wait()
            pltpu.sync_copy(bufs[r % NBUF], acc_sh.at[didx_v.at[r]], add=True)


def _count_body(oh128_hbm, c3_hbm, dst3_hbm, z128_hbm,
                cnt_out,
                cidx_v, didx_v, b0, b1, b2, cnt_sh,
                s0, s1, s2):
    bufs, sems = [b0, b1, b2], [s0, s1, s2]
    c = lax.axis_index("c")
    s = lax.axis_index("s")
    w = c * NS + s
    rows = pl.ds(s * RPT, RPT)
    pltpu.sync_copy(z128_hbm.at[rows], cnt_sh.at[rows])
    plsc.subcore_barrier()
    _gs_pipeline(oh128_hbm, c3_hbm, dst3_hbm, w, cidx_v, didx_v,
                 bufs, cnt_sh, sems)
    plsc.subcore_barrier()
    pltpu.sync_copy(cnt_sh.at[rows], cnt_out.at[c, rows])


_count = functools.partial(
    pl.kernel,
    _count_body,
    out_type=jax.ShapeDtypeStruct((NC, Np, D), jnp.float32),
    mesh=_mesh,
    scratch_types=[
        pltpu.VMEM((HC, ECH), jnp.int32),
        pltpu.VMEM((HC, ECH), jnp.int32),
        pltpu.VMEM((ECH, D), jnp.float32),
        pltpu.VMEM((ECH, D), jnp.float32),
        pltpu.VMEM((ECH, D), jnp.float32),
        pltpu.VMEM_SHARED((Np, D), jnp.float32),
        pltpu.SemaphoreType.DMA,
        pltpu.SemaphoreType.DMA,
        pltpu.SemaphoreType.DMA,
    ],
)()


def _agg_body(h_hbm, src3_hbm, dst3_hbm, z128_hbm,
              agg_out,
              sidx_v, didx_v, b0, b1, b2, agg_sh, s0, s1, s2):
    bufs, sems = [b0, b1, b2], [s0, s1, s2]
    c = lax.axis_index("c")
    s = lax.axis_index("s")
    w = c * NS + s
    rows = pl.ds(s * RPT, RPT)

    @pl.when(c == 0)
    def _():
        pltpu.sync_copy(h_hbm.at[rows], agg_sh.at[rows])

    @pl.when(c == 1)
    def _():
        pltpu.sync_copy(z128_hbm.at[rows], agg_sh.at[rows])

    plsc.subcore_barrier()
    _gs_pipeline(h_hbm, src3_hbm, dst3_hbm, w, sidx_v, didx_v,
                 bufs, agg_sh, sems)
    plsc.subcore_barrier()
    pltpu.sync_copy(agg_sh.at[rows], agg_out.at[c, rows])


_agg = functools.partial(
    pl.kernel,
    _agg_body,
    out_type=jax.ShapeDtypeStruct((NC, Np, D), jnp.float32),
    mesh=_mesh,
    scratch_types=[
        pltpu.VMEM((HC, ECH), jnp.int32),
        pltpu.VMEM((HC, ECH), jnp.int32),
        pltpu.VMEM((ECH, D), jnp.float32),
        pltpu.VMEM((ECH, D), jnp.float32),
        pltpu.VMEM((ECH, D), jnp.float32),
        pltpu.VMEM_SHARED((Np, D), jnp.float32),
        pltpu.SemaphoreType.DMA,
        pltpu.SemaphoreType.DMA,
        pltpu.SemaphoreType.DMA,
    ],
)()


RBLK = 1024


def _h0_body(idx_ref, comb_ref, out_ref):
    idx = idx_ref[...]
    onehot = jnp.where(idx == lax.broadcasted_iota(jnp.int32, (RBLK, 384), 1),
                       1.0, 0.0)
    out_ref[...] = jnp.dot(onehot, comb_ref[...],
                           preferred_element_type=jnp.float32)


def _h0_tc(x01p, comb384):
    return pl.pallas_call(
        _h0_body,
        grid=(Np // RBLK,),
        in_specs=[
            pl.BlockSpec((RBLK, 1), lambda i: (i, 0)),
            pl.BlockSpec((384, D), lambda i: (0, 0)),
        ],
        out_specs=pl.BlockSpec((RBLK, D), lambda i: (i, 0)),
        out_shape=jax.ShapeDtypeStruct((Np, D), jnp.float32),
    )(x01p, comb384)


def _mlp_body(relu, agg_ref, cnt_ref, cls_ref, w1_ref, b1_ref, w2_ref, b2_ref,
              g_ref, be_ref, sr_ref, out_ref):
    z = (agg_ref[0] + agg_ref[1]
         + jnp.dot(cnt_ref[0] + cnt_ref[1], cls_ref[...],
                   preferred_element_type=jnp.float32)
         + sr_ref[...])
    m = jnp.maximum(jnp.dot(z, w1_ref[...],
                            preferred_element_type=jnp.float32) + b1_ref[...],
                    0.0)
    o = jnp.dot(m, w2_ref[...], preferred_element_type=jnp.float32) + b2_ref[...]
    o = o * g_ref[...] + be_ref[...]
    out_ref[...] = jnp.maximum(o, 0.0) if relu else o


def _mlp(relu, agg2, cnt2, cls16, w1t, b1r, w2t, b2r, gr, ber, srr):
    grid = (Np // RBLK,)
    full = lambda shape: pl.BlockSpec(shape, lambda i: (0,) * len(shape))
    return pl.pallas_call(
        functools.partial(_mlp_body, relu),
        grid=grid,
        in_specs=[
            pl.BlockSpec((NC, RBLK, D), lambda i: (0, i, 0)),
            pl.BlockSpec((NC, RBLK, 16), lambda i: (0, i, 0)),
            full((16, D)),
            full((D, 2 * D)),
            full((1, 2 * D)),
            full((2 * D, D)),
            full((1, D)),
            full((1, D)),
            full((1, D)),
            full((1, D)),
        ],
        out_specs=pl.BlockSpec((RBLK, D), lambda i: (i, 0)),
        out_shape=jax.ShapeDtypeStruct((Np, D), jnp.float32),
    )(agg2, cnt2, cls16, w1t, b1r, w2t, b2r, gr, ber, srr)


def kernel(x, edge_index, edge_attr, xe1, xe2, ee1, ee2, W1, b1, W2, b2,
           gamma, beta):
    eps = 1e-5
    f32 = jnp.float32
    src = edge_index[0].astype(jnp.int32)
    dst = edge_index[1].astype(jnp.int32)
    pad_e = N + jnp.arange(Ep - E, dtype=jnp.int32) % (Np - N)
    src3 = jnp.concatenate([src, pad_e]).reshape(NW, NCHUNK, ECH)
    dst_p = jnp.concatenate([dst, pad_e])
    dst3 = dst_p.reshape(NW, NCHUNK, ECH)
    c_e = (edge_attr[:, 0] * 3 + edge_attr[:, 1]).astype(jnp.int32)
    i_d = jnp.arange(Ep - E, dtype=jnp.int32)
    pad_c = (16 * OHR * (i_d % NW) + 9 + (i_d // NW) % 7
             - 16 * OHR * (NW - 1))
    c3 = jnp.concatenate([c_e, pad_c]).reshape(NW, NCHUNK, ECH)
    c3 = (c3 + (jnp.arange(NW, dtype=jnp.int32) * 16 * OHR)[:, None, None]
          + (jnp.arange(NCHUNK, dtype=jnp.int32) % OHR * 16)[None, :, None])
    x01 = (x[:, 0] * 3 + x[:, 1]).astype(jnp.int32)
    x01p = jnp.concatenate([x01, jnp.zeros((Np - N,), jnp.int32)]
                           ).reshape(Np, 1)
    comb384 = jnp.concatenate(
        [(xe1[:, None, :] + xe2[None, :3, :]).reshape(-1, D),
         jnp.zeros((24, D), f32)], 0)
    oh128 = jnp.tile(jnp.eye(16, D, dtype=f32), (NW * OHR, 1))
    z128 = jnp.zeros((Np, D), f32)
    gsc = (gamma / jnp.sqrt(1.0 + eps)).astype(f32)

    cnt128 = _count(oh128, c3, dst3, z128)
    cnt2 = cnt128[:, :, :16]
    h = _h0_tc(x01p, comb384)
    for l in range(L):
        cls9 = (ee1[l, :3, None, :] + ee2[l, None, :3, :]).reshape(9, D)
        cls16 = jnp.concatenate([cls9, jnp.zeros((7, D), f32)], 0)
        srr = (ee1[l, 4] + ee2[l, 0]).reshape(1, D)
        agg2 = _agg(h, src3, dst3, z128)
        h = _mlp(l < L - 1, agg2, cnt2, cls16,
                 W1[l].T, b1[l].reshape(1, 2 * D),
                 W2[l].T, b2[l].reshape(1, D),
                 gsc[l].reshape(1, D), beta[l].reshape(1, D), srr)
    return h[:N]


# speedup vs baseline: 17.7416x; 1.0165x over previous
import functools

import jax
import jax.numpy as jnp
from jax import lax
from jax.experimental import pallas as pl
from jax.experimental.pallas import tpu as pltpu
from jax.experimental.pallas import tpu_sc as plsc

N = 10000
E = 320000
D = 128
L = 5

NC = 2
NS = 16
NW = NC * NS
Np = 10240
Ep = NW * Np
EPT = Ep // NW
ECH = 64
NCHUNK = EPT // ECH
NPT = Np // NW
NNCH = 64
NNCHUNK = NPT // NNCH
RPT = Np // NS
HC = NCHUNK // 2
NBUF = 3
OHR = 16

_mesh = plsc.VectorSubcoreMesh(core_axis_name="c", subcore_axis_name="s")


def _gs_pipeline(src_tab, idx3_hbm, didx3_hbm, w, sidx_v, didx_v,
                 bufs, acc_sh, sems):
    for p in range(2):
        pltpu.sync_copy(idx3_hbm.at[w, pl.ds(p * HC, HC)], sidx_v)
        pltpu.sync_copy(didx3_hbm.at[w, pl.ds(p * HC, HC)], didx_v)
        for k in range(NBUF - 1):
            pltpu.async_copy(src_tab.at[sidx_v.at[k]], bufs[k], sems[k])

        def body(jj, carry):
            for k in range(NBUF):
                j = NBUF * jj + k
                ka = (k + NBUF - 1) % NBUF

                @pl.when(j + NBUF - 1 < HC)
                def _():
                    pltpu.async_copy(src_tab.at[sidx_v.at[j + NBUF - 1]],
                                     bufs[ka], sems[ka])

                pltpu.make_async_copy(src_tab.at[sidx_v.at[j]], bufs[k],
                                      sems[k]).wait()
                pltpu.sync_copy(bufs[k], acc_sh.at[didx_v.at[j]], add=True)
            return carry

        lax.fori_loop(0, HC // NBUF, body, 0)
        for r in range(HC - HC % NBUF, HC):
            pltpu.make_async_copy(src_tab.at[sidx_v.at[r]], bufs[r % NBUF],
                                  sems[r % NBUF]).wait()
            pltpu.sync_copy(bufs[r % NBUF], acc_sh.at[didx_v.at[r]], add=True)


def _count_body(oh128_hbm, c3_hbm, dst3_hbm, z128_hbm,
                cnt_out,
                cidx_v, didx_v, b0, b1, b2, cnt_sh,
                s0, s1, s2):
    bufs, sems = [b0, b1, b2], [s0, s1, s2]
    c = lax.axis_index("c")
    s = lax.axis_index("s")
    w = c * NS + s
    rows = pl.ds(s * RPT, RPT)
    pltpu.sync_copy(z128_hbm.at[rows], cnt_sh.at[rows])
    plsc.subcore_barrier()
    _gs_pipeline(oh128_hbm, c3_hbm, dst3_hbm, w, cidx_v, didx_v,
                 bufs, cnt_sh, sems)
    plsc.subcore_barrier()
    pltpu.sync_copy(cnt_sh.at[rows], cnt_out.at[c, rows])


_count = functools.partial(
    pl.kernel,
    _count_body,
    out_type=jax.ShapeDtypeStruct((NC, Np, D), jnp.float32),
    mesh=_mesh,
    scratch_types=[
        pltpu.VMEM((HC, ECH), jnp.int32),
        pltpu.VMEM((HC, ECH), jnp.int32),
        pltpu.VMEM((ECH, D), jnp.float32),
        pltpu.VMEM((ECH, D), jnp.float32),
        pltpu.VMEM((ECH, D), jnp.float32),
        pltpu.VMEM_SHARED((Np, D), jnp.float32),
        pltpu.SemaphoreType.DMA,
        pltpu.SemaphoreType.DMA,
        pltpu.SemaphoreType.DMA,
    ],
)()


def _agg_body(h_hbm, src3_hbm, dst3_hbm, z128_hbm,
              agg_out,
              sidx_v, didx_v, b0, b1, b2, agg_sh, s0, s1, s2):
    bufs, sems = [b0, b1, b2], [s0, s1, s2]
    c = lax.axis_index("c")
    s = lax.axis_index("s")
    w = c * NS + s
    rows = pl.ds(s * RPT, RPT)

    @pl.when(c == 0)
    def _():
        pltpu.sync_copy(h_hbm.at[rows], agg_sh.at[rows])

    @pl.when(c == 1)
    def _():
        pltpu.sync_copy(z128_hbm.at[rows], agg_sh.at[rows])

    plsc.subcore_barrier()
    _gs_pipeline(h_hbm, src3_hbm, dst3_hbm, w, sidx_v, didx_v,
                 bufs, agg_sh, sems)
    plsc.subcore_barrier()
    pltpu.sync_copy(agg_sh.at[rows], agg_out.at[c, rows])


_agg = functools.partial(
    pl.kernel,
    _agg_body,
    out_type=jax.ShapeDtypeStruct((NC, Np, D), jnp.float32),
    mesh=_mesh,
    scratch_types=[
        pltpu.VMEM((HC, ECH), jnp.int32),
        pltpu.VMEM((HC, ECH), jnp.int32),
        pltpu.VMEM((ECH, D), jnp.float32),
        pltpu.VMEM((ECH, D), jnp.float32),
        pltpu.VMEM((ECH, D), jnp.float32),
        pltpu.VMEM_SHARED((Np, D), jnp.float32),
        pltpu.SemaphoreType.DMA,
        pltpu.SemaphoreType.DMA,
        pltpu.SemaphoreType.DMA,
    ],
)()


RBLK = 2048


def _h0_body(idx_ref, comb_ref, out_ref):
    idx = idx_ref[...]
    onehot = jnp.where(idx == lax.broadcasted_iota(jnp.int32, (RBLK, 384), 1),
                       1.0, 0.0)
    out_ref[...] = jnp.dot(onehot, comb_ref[...],
                           preferred_element_type=jnp.float32)


def _h0_tc(x01p, comb384):
    return pl.pallas_call(
        _h0_body,
        grid=(Np // RBLK,),
        in_specs=[
            pl.BlockSpec((RBLK, 1), lambda i: (i, 0)),
            pl.BlockSpec((384, D), lambda i: (0, 0)),
        ],
        out_specs=pl.BlockSpec((RBLK, D), lambda i: (i, 0)),
        out_shape=jax.ShapeDtypeStruct((Np, D), jnp.float32),
    )(x01p, comb384)


def _mlp_body(relu, agg_ref, cnt_ref, cls_ref, w1_ref, b1_ref, w2_ref, b2_ref,
              g_ref, be_ref, sr_ref, out_ref):
    z = (agg_ref[0] + agg_ref[1]
         + jnp.dot(cnt_ref[0] + cnt_ref[1], cls_ref[...],
                   preferred_element_type=jnp.float32)
         + sr_ref[...])
    m = jnp.maximum(jnp.dot(z, w1_ref[...],
                            preferred_element_type=jnp.float32) + b1_ref[...],
                    0.0)
    o = jnp.dot(m, w2_ref[...], preferred_element_type=jnp.float32) + b2_ref[...]
    o = o * g_ref[...] + be_ref[...]
    out_ref[...] = jnp.maximum(o, 0.0) if relu else o


def _mlp(relu, agg2, cnt2, cls16, w1t, b1r, w2t, b2r, gr, ber, srr):
    grid = (Np // RBLK,)
    full = lambda shape: pl.BlockSpec(shape, lambda i: (0,) * len(shape))
    return pl.pallas_call(
        functools.partial(_mlp_body, relu),
        grid=grid,
        in_specs=[
            pl.BlockSpec((NC, RBLK, D), lambda i: (0, i, 0)),
            pl.BlockSpec((NC, RBLK, 16), lambda i: (0, i, 0)),
            full((16, D)),
            full((D, 2 * D)),
            full((1, 2 * D)),
            full((2 * D, D)),
            full((1, D)),
            full((1, D)),
            full((1, D)),
            full((1, D)),
        ],
        out_specs=pl.BlockSpec((RBLK, D), lambda i: (i, 0)),
        out_shape=jax.ShapeDtypeStruct((Np, D), jnp.float32),
    )(agg2, cnt2, cls16, w1t, b1r, w2t, b2r, gr, ber, srr)


def kernel(x, edge_index, edge_attr, xe1, xe2, ee1, ee2, W1, b1, W2, b2,
           gamma, beta):
    eps = 1e-5
    f32 = jnp.float32
    src = edge_index[0].astype(jnp.int32)
    dst = edge_index[1].astype(jnp.int32)
    pad_e = N + jnp.arange(Ep - E, dtype=jnp.int32) % (Np - N)
    src3 = jnp.concatenate([src, pad_e]).reshape(NW, NCHUNK, ECH)
    dst_p = jnp.concatenate([dst, pad_e])
    dst3 = dst_p.reshape(NW, NCHUNK, ECH)
    c_e = (edge_attr[:, 0] * 3 + edge_attr[:, 1]).astype(jnp.int32)
    i_d = jnp.arange(Ep - E, dtype=jnp.int32)
    pad_c = (16 * OHR * (i_d % NW) + 9 + (i_d // NW) % 7
             - 16 * OHR * (NW - 1))
    c3 = jnp.concatenate([c_e, pad_c]).reshape(NW, NCHUNK, ECH)
    c3 = (c3 + (jnp.arange(NW, dtype=jnp.int32) * 16 * OHR)[:, None, None]
          + (jnp.arange(NCHUNK, dtype=jnp.int32) % OHR * 16)[None, :, None])
    x01 = (x[:, 0] * 3 + x[:, 1]).astype(jnp.int32)
    x01p = jnp.concatenate([x01, jnp.zeros((Np - N,), jnp.int32)]
                           ).reshape(Np, 1)
    comb384 = jnp.concatenate(
        [(xe1[:, None, :] + xe2[None, :3, :]).reshape(-1, D),
         jnp.zeros((24, D), f32)], 0)
    oh128 = jnp.tile(jnp.eye(16, D, dtype=f32), (NW * OHR, 1))
    z128 = jnp.zeros((Np, D), f32)
    gsc = (gamma / jnp.sqrt(1.0 + eps)).astype(f32)

    cnt128 = _count(oh128, c3, dst3, z128)
    cnt2 = cnt128[:, :, :16]
    h = _h0_tc(x01p, comb384)
    for l in range(L):
        cls9 = (ee1[l, :3, None, :] + ee2[l, None, :3, :]).reshape(9, D)
        cls16 = jnp.concatenate([cls9, jnp.zeros((7, D), f32)], 0)
        srr = (ee1[l, 4] + ee2[l, 0]).reshape(1, D)
        agg2 = _agg(h, src3, dst3, z128)
        h = _mlp(l < L - 1, agg2, cnt2, cls16,
                 W1[l].T, b1[l].reshape(1, 2 * D),
                 W2[l].T, b2[l].reshape(1, D),
                 gsc[l].reshape(1, D), beta[l].reshape(1, D), srr)
    return h[:N]


# speedup vs baseline: 17.9185x; 1.0100x over previous
import functools

import jax
import jax.numpy as jnp
from jax import lax
from jax.experimental import pallas as pl
from jax.experimental.pallas import tpu as pltpu
from jax.experimental.pallas import tpu_sc as plsc

N = 10000
E = 320000
D = 128
L = 5

NC = 2
NS = 16
NW = NC * NS
Np = 10240
Ep = NW * Np
EPT = Ep // NW
ECH = 64
NCHUNK = EPT // ECH
NPT = Np // NW
NNCH = 64
NNCHUNK = NPT // NNCH
RPT = Np // NS
HC = NCHUNK // 2
NBUF = 3
OHR = 16

_mesh = plsc.VectorSubcoreMesh(core_axis_name="c", subcore_axis_name="s")


def _gs_pipeline(src_tab, idx3_hbm, didx3_hbm, w, sidx_v, didx_v,
                 bufs, acc_sh, sems):
    for p in range(2):
        pltpu.sync_copy(idx3_hbm.at[w, pl.ds(p * HC, HC)], sidx_v)
        pltpu.sync_copy(didx3_hbm.at[w, pl.ds(p * HC, HC)], didx_v)
        for k in range(NBUF - 1):
            pltpu.async_copy(src_tab.at[sidx_v.at[k]], bufs[k], sems[k])

        def body(jj, carry):
            for k in range(NBUF):
                j = NBUF * jj + k
                ka = (k + NBUF - 1) % NBUF

                @pl.when(j + NBUF - 1 < HC)
                def _():
                    pltpu.async_copy(src_tab.at[sidx_v.at[j + NBUF - 1]],
                                     bufs[ka], sems[ka])

                pltpu.make_async_copy(src_tab.at[sidx_v.at[j]], bufs[k],
                                      sems[k]).wait()
                pltpu.sync_copy(bufs[k], acc_sh.at[didx_v.at[j]], add=True)
            return carry

        lax.fori_loop(0, HC // NBUF, body, 0)
        for r in range(HC - HC % NBUF, HC):
            pltpu.make_async_copy(src_tab.at[sidx_v.at[r]], bufs[r % NBUF],
                                  sems[r % NBUF]).wait()
            pltpu.sync_copy(bufs[r % NBUF], acc_sh.at[didx_v.at[r]], add=True)


def _count_body(oh128_hbm, c3_hbm, dst3_hbm, z128_hbm,
                cnt_out,
                cidx_v, didx_v, b0, b1, b2, cnt_sh,
                s0, s1, s2):
    bufs, sems = [b0, b1, b2], [s0, s1, s2]
    c = lax.axis_index("c")
    s = lax.axis_index("s")
    w = c * NS + s
    rows = pl.ds(s * RPT, RPT)
    pltpu.sync_copy(z128_hbm.at[rows], cnt_sh.at[rows])
    plsc.subcore_barrier()
    _gs_pipeline(oh128_hbm, c3_hbm, dst3_hbm, w, cidx_v, didx_v,
                 bufs, cnt_sh, sems)
    plsc.subcore_barrier()
    pltpu.sync_copy(cnt_sh.at[rows], cnt_out.at[c, rows])


_count = functools.partial(
    pl.kernel,
    _count_body,
    out_type=jax.ShapeDtypeStruct((NC, Np, D), jnp.float32),
    mesh=_mesh,
    scratch_types=[
        pltpu.VMEM((HC, ECH), jnp.int32),
        pltpu.VMEM((HC, ECH), jnp.int32),
        pltpu.VMEM((ECH, D), jnp.float32),
        pltpu.VMEM((ECH, D), jnp.float32),
        pltpu.VMEM((ECH, D), jnp.float32),
        pltpu.VMEM_SHARED((Np, D), jnp.float32),
        pltpu.SemaphoreType.DMA,
        pltpu.SemaphoreType.DMA,
        pltpu.SemaphoreType.DMA,
    ],
)()


def _agg_body(h_hbm, src3_hbm, dst3_hbm, z128_hbm,
              agg_out,
              sidx_v, didx_v, b0, b1, b2, agg_sh, s0, s1, s2):
    bufs, sems = [b0, b1, b2], [s0, s1, s2]
    c = lax.axis_index("c")
    s = lax.axis_index("s")
    w = c * NS + s
    rows = pl.ds(s * RPT, RPT)

    @pl.when(c == 0)
    def _():
        pltpu.sync_copy(h_hbm.at[rows], agg_sh.at[rows])

    @pl.when(c == 1)
    def _():
        pltpu.sync_copy(z128_hbm.at[rows], agg_sh.at[rows])

    plsc.subcore_barrier()
    _gs_pipeline(h_hbm, src3_hbm, dst3_hbm, w, sidx_v, didx_v,
                 bufs, agg_sh, sems)
    plsc.subcore_barrier()
    pltpu.sync_copy(agg_sh.at[rows], agg_out.at[c, rows])


_agg = functools.partial(
    pl.kernel,
    _agg_body,
    out_type=jax.ShapeDtypeStruct((NC, Np, D), jnp.float32),
    mesh=_mesh,
    scratch_types=[
        pltpu.VMEM((HC, ECH), jnp.int32),
        pltpu.VMEM((HC, ECH), jnp.int32),
        pltpu.VMEM((ECH, D), jnp.float32),
        pltpu.VMEM((ECH, D), jnp.float32),
        pltpu.VMEM((ECH, D), jnp.float32),
        pltpu.VMEM_SHARED((Np, D), jnp.float32),
        pltpu.SemaphoreType.DMA,
        pltpu.SemaphoreType.DMA,
        pltpu.SemaphoreType.DMA,
    ],
)()


RBLK = 5120


def _h0_body(idx_ref, comb_ref, out_ref):
    idx = idx_ref[...]
    onehot = jnp.where(idx == lax.broadcasted_iota(jnp.int32, (RBLK, 384), 1),
                       1.0, 0.0)
    out_ref[...] = jnp.dot(onehot, comb_ref[...],
                           preferred_element_type=jnp.float32)


def _h0_tc(x01p, comb384):
    return pl.pallas_call(
        _h0_body,
        grid=(Np // RBLK,),
        in_specs=[
            pl.BlockSpec((RBLK, 1), lambda i: (i, 0)),
            pl.BlockSpec((384, D), lambda i: (0, 0)),
        ],
        out_specs=pl.BlockSpec((RBLK, D), lambda i: (i, 0)),
        out_shape=jax.ShapeDtypeStruct((Np, D), jnp.float32),
    )(x01p, comb384)


def _mlp_body(relu, agg_ref, cnt_ref, cls_ref, w1_ref, b1_ref, w2_ref, b2_ref,
              g_ref, be_ref, sr_ref, out_ref):
    z = (agg_ref[0] + agg_ref[1]
         + jnp.dot(cnt_ref[0] + cnt_ref[1], cls_ref[...],
                   preferred_element_type=jnp.float32)
         + sr_ref[...])
    m = jnp.maximum(jnp.dot(z, w1_ref[...],
                            preferred_element_type=jnp.float32) + b1_ref[...],
                    0.0)
    o = jnp.dot(m, w2_ref[...], preferred_element_type=jnp.float32) + b2_ref[...]
    o = o * g_ref[...] + be_ref[...]
    out_ref[...] = jnp.maximum(o, 0.0) if relu else o


def _mlp(relu, agg2, cnt2, cls16, w1t, b1r, w2t, b2r, gr, ber, srr):
    grid = (Np // RBLK,)
    full = lambda shape: pl.BlockSpec(shape, lambda i: (0,) * len(shape))
    return pl.pallas_call(
        functools.partial(_mlp_body, relu),
        grid=grid,
        in_specs=[
            pl.BlockSpec((NC, RBLK, D), lambda i: (0, i, 0)),
            pl.BlockSpec((NC, RBLK, 16), lambda i: (0, i, 0)),
            full((16, D)),
            full((D, 2 * D)),
            full((1, 2 * D)),
            full((2 * D, D)),
            full((1, D)),
            full((1, D)),
            full((1, D)),
            full((1, D)),
        ],
        out_specs=pl.BlockSpec((RBLK, D), lambda i: (i, 0)),
        out_shape=jax.ShapeDtypeStruct((Np, D), jnp.float32),
    )(agg2, cnt2, cls16, w1t, b1r, w2t, b2r, gr, ber, srr)


def kernel(x, edge_index, edge_attr, xe1, xe2, ee1, ee2, W1, b1, W2, b2,
           gamma, beta):
    eps = 1e-5
    f32 = jnp.float32
    src = edge_index[0].astype(jnp.int32)
    dst = edge_index[1].astype(jnp.int32)
    pad_e = N + jnp.arange(Ep - E, dtype=jnp.int32) % (Np - N)
    src3 = jnp.concatenate([src, pad_e]).reshape(NW, NCHUNK, ECH)
    dst_p = jnp.concatenate([dst, pad_e])
    dst3 = dst_p.reshape(NW, NCHUNK, ECH)
    c_e = (edge_attr[:, 0] * 3 + edge_attr[:, 1]).astype(jnp.int32)
    i_d = jnp.arange(Ep - E, dtype=jnp.int32)
    pad_c = (16 * OHR * (i_d % NW) + 9 + (i_d // NW) % 7
             - 16 * OHR * (NW - 1))
    c3 = jnp.concatenate([c_e, pad_c]).reshape(NW, NCHUNK, ECH)
    c3 = (c3 + (jnp.arange(NW, dtype=jnp.int32) * 16 * OHR)[:, None, None]
          + (jnp.arange(NCHUNK, dtype=jnp.int32) % OHR * 16)[None, :, None])
    x01 = (x[:, 0] * 3 + x[:, 1]).astype(jnp.int32)
    x01p = jnp.concatenate([x01, jnp.zeros((Np - N,), jnp.int32)]
                           ).reshape(Np, 1)
    comb384 = jnp.concatenate(
        [(xe1[:, None, :] + xe2[None, :3, :]).reshape(-1, D),
         jnp.zeros((24, D), f32)], 0)
    oh128 = jnp.tile(jnp.eye(16, D, dtype=f32), (NW * OHR, 1))
    z128 = jnp.zeros((Np, D), f32)
    gsc = (gamma / jnp.sqrt(1.0 + eps)).astype(f32)

    cnt128 = _count(oh128, c3, dst3, z128)
    cnt2 = cnt128[:, :, :16]
    h = _h0_tc(x01p, comb384)
    for l in range(L):
        cls9 = (ee1[l, :3, None, :] + ee2[l, None, :3, :]).reshape(9, D)
        cls16 = jnp.concatenate([cls9, jnp.zeros((7, D), f32)], 0)
        srr = (ee1[l, 4] + ee2[l, 0]).reshape(1, D)
        agg2 = _agg(h, src3, dst3, z128)
        h = _mlp(l < L - 1, agg2, cnt2, cls16,
                 W1[l].T, b1[l].reshape(1, 2 * D),
                 W2[l].T, b2[l].reshape(1, D),
                 gsc[l].reshape(1, D), beta[l].reshape(1, D), srr)
    return h[:N]
